# Initial kernel scaffold; baseline (speedup 1.0000x reference)
import jax
import jax.numpy as jnp
from jax.experimental import pallas as pl


def kernel(x_scalar, x_spherical, rbf, rsh, W1, b1, W2, b2, rbf_w, ln_g, ln_b, o3_w, o3_b, tp_w, edge_index):
    raise NotImplementedError("write your pallas kernel here")


# speedup vs baseline: 1.4044x; 1.4044x over previous
import functools
from math import factorial

import jax
import jax.numpy as jnp
import numpy as np
from jax.experimental import pallas as pl
from jax.experimental.pallas import tpu as pltpu

NODE_DIM = 128
NUM_BASIS = 20
IRREPS = [(128, 0), (64, 1), (32, 2)]
SPH = [(1, 0), (1, 1), (1, 2)]
NUM_IRREPS = sum(m for m, _ in IRREPS)
EDGE_DIM = sum(m * (2 * l + 1) for m, l in IRREPS)
SPH_DIM = sum(m * (2 * l + 1) for m, l in SPH)
HIDDEN = NODE_DIM + NUM_IRREPS
N_NODES = 10000
N_EDGES = 160000
N_PAD = 10240

NODE_BLK = 128
EDGE_BLK = 256


def _su2_cg(j1, j2, j3, m1, m2, m3):
    if m3 != m1 + m2:
        return 0.0
    vmin = int(max(-j1 + j2 + m3, -j1 + m1, 0))
    vmax = int(min(j2 + j3 + m1, j3 - j1 + j2, j3 + m3))

    def f(n):
        return float(factorial(round(n)))

    C = ((2 * j3 + 1) * f(j3 + j1 - j2) * f(j3 - j1 + j2) * f(j1 + j2 - j3) / f(j1 + j2 + j3 + 1)
         * f(j3 + m3) * f(j3 - m3) / (f(j1 - m1) * f(j1 + m1) * f(j2 - m2) * f(j2 + m2))) ** 0.5
    S = 0.0
    for v in range(vmin, vmax + 1):
        S += (-1.0) ** (v + j2 + m2) / f(v) * f(j2 + j3 + m1 - v) * f(j1 - m1 + v) / (
            f(j3 - j1 + j2 - v) * f(j3 + m3 - v) * f(v + j1 - j2 - m3))
    return C * S


def _su2_cg_tensor(l1, l2, l3):
    C = np.zeros((2 * l1 + 1, 2 * l2 + 1, 2 * l3 + 1))
    for m1 in range(-l1, l1 + 1):
        for m2 in range(-l2, l2 + 1):
            m3 = m1 + m2
            if abs(m3) <= l3:
                C[m1 + l1, m2 + l2, m3 + l3] = _su2_cg(l1, l2, l3, m1, m2, m3)
    return C


def _q_mat(l):
    q = np.zeros((2 * l + 1, 2 * l + 1), dtype=complex)
    for m in range(-l, 0):
        q[l + m, l + abs(m)] = 1 / 2 ** 0.5
        q[l + m, l - abs(m)] = -1j / 2 ** 0.5
    q[l, l] = 1.0
    for m in range(1, l + 1):
        q[l + m, l + abs(m)] = (-1) ** m / 2 ** 0.5
        q[l + m, l - abs(m)] = 1j * (-1) ** m / 2 ** 0.5
    return (-1j) ** l * q


def _w3j(l1, l2, l3):
    C = _su2_cg_tensor(l1, l2, l3).astype(complex)
    Q1, Q2, Q3 = _q_mat(l1), _q_mat(l2), _q_mat(l3)
    C = np.einsum('ij,kl,mn,ikm->jln', Q1, Q2, np.conj(Q3), C)
    re, im = np.real(C), np.imag(C)
    C = re if np.abs(re).sum() >= np.abs(im).sum() else im
    n = np.linalg.norm(C)
    return C / n if n > 0 else C


_PATHS = []
for _i1, (_m1, _l1) in enumerate(IRREPS):
    for _i2, (_m2, _l2) in enumerate(SPH):
        for _io, (_mo, _l3) in enumerate(IRREPS):
            if abs(_l1 - _l2) <= _l3 <= _l1 + _l2:
                _PATHS.append((_i1, _i2, _io))
_FAN_IN = [0] * len(IRREPS)
for (_i1, _i2, _io) in _PATHS:
    _FAN_IN[_io] += IRREPS[_i1][0] * SPH[_i2][0]
_W3J = {}
for (_i1, _i2, _io) in _PATHS:
    _k = (IRREPS[_i1][1], SPH[_i2][1], IRREPS[_io][1])
    if _k not in _W3J:
        _W3J[_k] = _w3j(*_k)

_YW = {0: 0, 1: 0, 2: 0}
_COMBO = []
for (_i1, _i2, _io) in _PATHS:
    _m1, _l1 = IRREPS[_i1]
    _, _l2 = SPH[_i2]
    _mo, _l3 = IRREPS[_io]
    _cg = _W3J[(_l1, _l2, _l3)]
    _terms = {}
    for _i in range(2 * _l1 + 1):
        for _j in range(2 * _l2 + 1):
            for _kk in range(2 * _l3 + 1):
                _c = _cg[_i, _j, _kk]
                if abs(_c) > 1e-12:
                    _terms.setdefault((_i, _kk), []).append((_j, float(_c)))
    _COMBO.append((_l1, _l2, _io, _YW[_l1], _mo, _terms))
    _YW[_l1] += _mo

_RBASE = {0: 0, 1: 1, 2: 4}


def _prep_tp_weights(tp_w):
    groups = {0: [], 1: [], 2: []}
    off = 0
    for (i1, i2, io) in _PATHS:
        m1, l1 = IRREPS[i1]
        mo, l3 = IRREPS[io]
        w = tp_w[off:off + m1 * mo].reshape(m1, mo)
        off += m1 * mo
        alpha = (2 * l3 + 1) ** 0.5 / _FAN_IN[io] ** 0.5
        groups[l1].append(w * alpha)
    return (jnp.concatenate(groups[0], axis=1),
            jnp.concatenate(groups[1], axis=1),
            jnp.concatenate(groups[2], axis=1))


def _node_body(xs_ref, xsp_ref, w1_ref, b1_ref, w2_ref, b2_ref, lng_ref,
               lnb_ref, colw_ref, colb_ref, sin_ref, sout_ref, sphn_ref):
    f32 = jnp.float32
    bf = jnp.bfloat16
    x = xs_ref[...]
    mu = jnp.mean(x, axis=1, keepdims=True)
    xc = x - mu
    var = jnp.mean(xc * xc, axis=1, keepdims=True)
    sin = xc / jnp.sqrt(var + 1e-5) * lng_ref[...] + lnb_ref[...]
    sin_ref[...] = sin
    h = jnp.dot(sin.astype(bf), w1_ref[...].astype(bf),
                preferred_element_type=f32) + b1_ref[...]
    h = h * jax.nn.sigmoid(h)
    sout_ref[...] = jnp.dot(h.astype(bf), w2_ref[...].astype(bf),
                            preferred_element_type=f32) + b2_ref[...]

    sp = xsp_ref[...]
    v = sp[:, :128]
    mu0 = jnp.mean(v, axis=1, keepdims=True)
    v = v - mu0
    o0 = v / jnp.sqrt(jnp.mean(v * v, axis=1, keepdims=True) + 1e-5)
    blk1 = sp[:, 128:320]
    n1 = jnp.sum(blk1 * blk1, axis=1, keepdims=True) * (1.0 / 64.0)
    o1 = blk1 / jnp.sqrt(n1 + 1e-5)
    blk2 = sp[:, 320:480]
    n2 = jnp.sum(blk2 * blk2, axis=1, keepdims=True) * (1.0 / 32.0)
    o2 = blk2 / jnp.sqrt(n2 + 1e-5)
    out = jnp.concatenate([o0, o1, o2], axis=1) * colw_ref[...] + colb_ref[...]
    sphn_ref[...] = out


def _node_stage(x_scalar, x_spherical, W1, b1, W2, b2, ln_g, ln_b, col_w, col_b):
    nblk = N_PAD // NODE_BLK
    full = lambda shape: pl.BlockSpec(shape, lambda i: (0, 0))
    row = lambda n: pl.BlockSpec((NODE_BLK, n), lambda i: (i, 0))
    return pl.pallas_call(
        _node_body,
        grid=(nblk,),
        in_specs=[row(NODE_DIM), row(EDGE_DIM),
                  full((NODE_DIM, NODE_DIM)), full((1, NODE_DIM)),
                  full((NODE_DIM, HIDDEN)), full((1, HIDDEN)),
                  full((1, NODE_DIM)), full((1, NODE_DIM)),
                  full((1, EDGE_DIM)), full((1, EDGE_DIM))],
        out_specs=[row(NODE_DIM), row(HIDDEN), row(EDGE_DIM)],
        out_shape=[jax.ShapeDtypeStruct((N_PAD, NODE_DIM), jnp.float32),
                   jax.ShapeDtypeStruct((N_PAD, HIDDEN), jnp.float32),
                   jax.ShapeDtypeStruct((N_PAD, EDGE_DIM), jnp.float32)],
    )(x_scalar, x_spherical, W1, b1.reshape(1, -1), W2, b2.reshape(1, -1),
      ln_g.reshape(1, -1), ln_b.reshape(1, -1), col_w, col_b)


def _edge_body(gsc_ref, gsp_ref, rbf_ref, rsh_ref, rbfw_ref, w0_ref, w1_ref,
               w2_ref, msc_ref, ms0_ref, ms1_ref, ms2_ref):
    f32 = jnp.float32
    bf = jnp.bfloat16
    fw = jnp.dot(rbf_ref[...].astype(bf), rbfw_ref[...].astype(bf),
                 preferred_element_type=f32)
    fo = gsc_ref[...] * fw
    gate = fo[:, :NUM_IRREPS]
    msc_ref[...] = fo[:, NUM_IRREPS:]

    gsp = gsp_ref[...]
    g0 = gate[:, :128]
    g1 = gate[:, 128:192]
    g2 = gate[:, 192:224]
    x0 = (gsp[:, :128] * g0).astype(bf)
    x1 = [(gsp[:, 128 + 64 * i:128 + 64 * (i + 1)] * g1).astype(bf)
          for i in range(3)]
    x2 = [(gsp[:, 320 + 32 * i:320 + 32 * (i + 1)] * g2).astype(bf)
          for i in range(5)]
    W0 = w0_ref[...].astype(bf)
    W1 = w1_ref[...].astype(bf)
    W2 = w2_ref[...].astype(bf)
    Y = {0: [jnp.dot(x0, W0, preferred_element_type=f32)],
         1: [jnp.dot(x, W1, preferred_element_type=f32) for x in x1],
         2: [jnp.dot(x, W2, preferred_element_type=f32) for x in x2]}

    rsh = rsh_ref[...]
    acc = {0: [None], 1: [None] * 3, 2: [None] * 5}
    for (l1, l2, io, yoff, mo, terms) in _COMBO:
        rbase = _RBASE[l2]
        for (i, k), jl in sorted(terms.items()):
            kv = None
            for (j, c) in jl:
                t = c * rsh[:, rbase + j:rbase + j + 1]
                kv = t if kv is None else kv + t
            contrib = kv * Y[l1][i][:, yoff:yoff + mo]
            acc[io][k] = contrib if acc[io][k] is None else acc[io][k] + contrib
    sph = jnp.concatenate(acc[0] + acc[1] + acc[2], axis=1)
    ms0_ref[...] = sph[:, 0:160]
    ms1_ref[...] = sph[:, 160:320]
    ms2_ref[...] = sph[:, 320:480]


def _edge_stage(gsc, gsp, rbf, rsh, rbf_w, W0, W1, W2):
    nblk = N_EDGES // EDGE_BLK
    full = lambda shape: pl.BlockSpec(shape, lambda i: (0, 0))
    row = lambda n: pl.BlockSpec((EDGE_BLK, n), lambda i: (i, 0))
    return pl.pallas_call(
        _edge_body,
        grid=(nblk,),
        in_specs=[row(HIDDEN), row(EDGE_DIM), row(NUM_BASIS), row(SPH_DIM),
                  full((NUM_BASIS, HIDDEN)), full((128, 224)),
                  full((64, 384)), full((32, 352))],
        out_specs=[row(NODE_DIM), row(160), row(160), row(160)],
        out_shape=[jax.ShapeDtypeStruct((N_EDGES, NODE_DIM), jnp.float32),
                   jax.ShapeDtypeStruct((N_EDGES, 160), jnp.float32),
                   jax.ShapeDtypeStruct((N_EDGES, 160), jnp.float32),
                   jax.ShapeDtypeStruct((N_EDGES, 160), jnp.float32)],
    )(gsc, gsp, rbf, rsh, rbf_w, W0, W1, W2)


def _to_imajor(sph):
    n = sph.shape[0]
    l1 = sph[:, 128:320].reshape(n, 64, 3).transpose(0, 2, 1).reshape(n, 192)
    l2 = sph[:, 320:480].reshape(n, 32, 5).transpose(0, 2, 1).reshape(n, 160)
    return jnp.concatenate([sph[:, :128], l1, l2], axis=1)


def _from_imajor(sph):
    n = sph.shape[0]
    l1 = sph[:, 128:320].reshape(n, 3, 64).transpose(0, 2, 1).reshape(n, 192)
    l2 = sph[:, 320:480].reshape(n, 5, 32).transpose(0, 2, 1).reshape(n, 160)
    return jnp.concatenate([sph[:, :128], l1, l2], axis=1)


def kernel(x_scalar, x_spherical, rbf, rsh, W1, b1, W2, b2, rbf_w, ln_g, ln_b,
           o3_w, o3_b, tp_w, edge_index):
    col_w = jnp.concatenate([
        o3_w[:128],
        jnp.repeat(o3_w[128:192], 3),
        jnp.repeat(o3_w[192:224], 5)]).reshape(1, EDGE_DIM)
    col_b = jnp.concatenate(
        [o3_b, jnp.zeros((EDGE_DIM - 128,), jnp.float32)]).reshape(1, EDGE_DIM)

    xs = jnp.pad(x_scalar, ((0, N_PAD - N_NODES), (0, 0)))
    xsp = jnp.pad(x_spherical, ((0, N_PAD - N_NODES), (0, 0)))
    scalar_in, scalar_out, sph_in = _node_stage(
        xs, xsp, W1, b1, W2, b2, ln_g, ln_b, col_w, col_b)
    sph_in_im = _to_imajor(sph_in)

    W0c, W1c, W2c = _prep_tp_weights(tp_w)

    src = edge_index[1]
    dst = edge_index[0]
    gsc = scalar_out[src]
    gsp = sph_in_im[src]

    msc, ms0, ms1, ms2 = _edge_stage(gsc, gsp, rbf, rsh, rbf_w, W0c, W1c, W2c)
    msph = jnp.concatenate([ms0, ms1, ms2], axis=1)

    new_scalar = scalar_in[:N_NODES].at[dst].add(msc)
    new_sph_im = sph_in_im[:N_NODES].at[dst].add(msph)
    return new_scalar, _from_imajor(new_sph_im)


# speedup vs baseline: 1.5285x; 1.0884x over previous
import functools
from math import factorial

import jax
import jax.numpy as jnp
import numpy as np
from jax.experimental import pallas as pl
from jax.experimental.pallas import tpu as pltpu
from jax.experimental.pallas import tpu_sc as plsc

NODE_DIM = 128
NUM_BASIS = 20
IRREPS = [(128, 0), (64, 1), (32, 2)]
SPH = [(1, 0), (1, 1), (1, 2)]
NUM_IRREPS = sum(m for m, _ in IRREPS)
EDGE_DIM = sum(m * (2 * l + 1) for m, l in IRREPS)
SPH_DIM = sum(m * (2 * l + 1) for m, l in SPH)
HIDDEN = NODE_DIM + NUM_IRREPS
N_NODES = 10000
N_EDGES = 160000
N_PAD = 10240

NODE_BLK = 128
EDGE_BLK = 256


def _su2_cg(j1, j2, j3, m1, m2, m3):
    if m3 != m1 + m2:
        return 0.0
    vmin = int(max(-j1 + j2 + m3, -j1 + m1, 0))
    vmax = int(min(j2 + j3 + m1, j3 - j1 + j2, j3 + m3))

    def f(n):
        return float(factorial(round(n)))

    C = ((2 * j3 + 1) * f(j3 + j1 - j2) * f(j3 - j1 + j2) * f(j1 + j2 - j3) / f(j1 + j2 + j3 + 1)
         * f(j3 + m3) * f(j3 - m3) / (f(j1 - m1) * f(j1 + m1) * f(j2 - m2) * f(j2 + m2))) ** 0.5
    S = 0.0
    for v in range(vmin, vmax + 1):
        S += (-1.0) ** (v + j2 + m2) / f(v) * f(j2 + j3 + m1 - v) * f(j1 - m1 + v) / (
            f(j3 - j1 + j2 - v) * f(j3 + m3 - v) * f(v + j1 - j2 - m3))
    return C * S


def _su2_cg_tensor(l1, l2, l3):
    C = np.zeros((2 * l1 + 1, 2 * l2 + 1, 2 * l3 + 1))
    for m1 in range(-l1, l1 + 1):
        for m2 in range(-l2, l2 + 1):
            m3 = m1 + m2
            if abs(m3) <= l3:
                C[m1 + l1, m2 + l2, m3 + l3] = _su2_cg(l1, l2, l3, m1, m2, m3)
    return C


def _q_mat(l):
    q = np.zeros((2 * l + 1, 2 * l + 1), dtype=complex)
    for m in range(-l, 0):
        q[l + m, l + abs(m)] = 1 / 2 ** 0.5
        q[l + m, l - abs(m)] = -1j / 2 ** 0.5
    q[l, l] = 1.0
    for m in range(1, l + 1):
        q[l + m, l + abs(m)] = (-1) ** m / 2 ** 0.5
        q[l + m, l - abs(m)] = 1j * (-1) ** m / 2 ** 0.5
    return (-1j) ** l * q


def _w3j(l1, l2, l3):
    C = _su2_cg_tensor(l1, l2, l3).astype(complex)
    Q1, Q2, Q3 = _q_mat(l1), _q_mat(l2), _q_mat(l3)
    C = np.einsum('ij,kl,mn,ikm->jln', Q1, Q2, np.conj(Q3), C)
    re, im = np.real(C), np.imag(C)
    C = re if np.abs(re).sum() >= np.abs(im).sum() else im
    n = np.linalg.norm(C)
    return C / n if n > 0 else C


_PATHS = []
for _i1, (_m1, _l1) in enumerate(IRREPS):
    for _i2, (_m2, _l2) in enumerate(SPH):
        for _io, (_mo, _l3) in enumerate(IRREPS):
            if abs(_l1 - _l2) <= _l3 <= _l1 + _l2:
                _PATHS.append((_i1, _i2, _io))
_FAN_IN = [0] * len(IRREPS)
for (_i1, _i2, _io) in _PATHS:
    _FAN_IN[_io] += IRREPS[_i1][0] * SPH[_i2][0]
_W3J = {}
for (_i1, _i2, _io) in _PATHS:
    _k = (IRREPS[_i1][1], SPH[_i2][1], IRREPS[_io][1])
    if _k not in _W3J:
        _W3J[_k] = _w3j(*_k)

_YW = {0: 0, 1: 0, 2: 0}
_COMBO = []
for (_i1, _i2, _io) in _PATHS:
    _m1, _l1 = IRREPS[_i1]
    _, _l2 = SPH[_i2]
    _mo, _l3 = IRREPS[_io]
    _cg = _W3J[(_l1, _l2, _l3)]
    _terms = {}
    for _i in range(2 * _l1 + 1):
        for _j in range(2 * _l2 + 1):
            for _kk in range(2 * _l3 + 1):
                _c = _cg[_i, _j, _kk]
                if abs(_c) > 1e-12:
                    _terms.setdefault((_i, _kk), []).append((_j, float(_c)))
    _COMBO.append((_l1, _l2, _io, _YW[_l1], _mo, _terms))
    _YW[_l1] += _mo

_RBASE = {0: 0, 1: 1, 2: 4}


def _prep_tp_weights(tp_w):
    groups = {0: [], 1: [], 2: []}
    off = 0
    for (i1, i2, io) in _PATHS:
        m1, l1 = IRREPS[i1]
        mo, l3 = IRREPS[io]
        w = tp_w[off:off + m1 * mo].reshape(m1, mo)
        off += m1 * mo
        alpha = (2 * l3 + 1) ** 0.5 / _FAN_IN[io] ** 0.5
        groups[l1].append(w * alpha)
    return (jnp.concatenate(groups[0], axis=1),
            jnp.concatenate(groups[1], axis=1),
            jnp.concatenate(groups[2], axis=1))


def _node_body(xs_ref, xsp_ref, w1_ref, b1_ref, w2_ref, b2_ref, lng_ref,
               lnb_ref, colw_ref, colb_ref, sin_ref, sout_ref, sphn_ref):
    f32 = jnp.float32
    bf = jnp.bfloat16
    x = xs_ref[...]
    mu = jnp.mean(x, axis=1, keepdims=True)
    xc = x - mu
    var = jnp.mean(xc * xc, axis=1, keepdims=True)
    sin = xc / jnp.sqrt(var + 1e-5) * lng_ref[...] + lnb_ref[...]
    sin_ref[...] = sin
    h = jnp.dot(sin.astype(bf), w1_ref[...].astype(bf),
                preferred_element_type=f32) + b1_ref[...]
    h = h * jax.nn.sigmoid(h)
    sout_ref[...] = jnp.dot(h.astype(bf), w2_ref[...].astype(bf),
                            preferred_element_type=f32) + b2_ref[...]

    sp = xsp_ref[...]
    v = sp[:, :128]
    mu0 = jnp.mean(v, axis=1, keepdims=True)
    v = v - mu0
    o0 = v / jnp.sqrt(jnp.mean(v * v, axis=1, keepdims=True) + 1e-5)
    blk1 = sp[:, 128:320]
    n1 = jnp.sum(blk1 * blk1, axis=1, keepdims=True) * (1.0 / 64.0)
    o1 = blk1 / jnp.sqrt(n1 + 1e-5)
    blk2 = sp[:, 320:480]
    n2 = jnp.sum(blk2 * blk2, axis=1, keepdims=True) * (1.0 / 32.0)
    o2 = blk2 / jnp.sqrt(n2 + 1e-5)
    out = jnp.concatenate([o0, o1, o2], axis=1) * colw_ref[...] + colb_ref[...]
    sphn_ref[...] = out


def _node_stage(x_scalar, x_spherical, W1, b1, W2, b2, ln_g, ln_b, col_w, col_b):
    nblk = N_PAD // NODE_BLK
    full = lambda shape: pl.BlockSpec(shape, lambda i: (0, 0))
    row = lambda n: pl.BlockSpec((NODE_BLK, n), lambda i: (i, 0))
    return pl.pallas_call(
        _node_body,
        grid=(nblk,),
        in_specs=[row(NODE_DIM), row(EDGE_DIM),
                  full((NODE_DIM, NODE_DIM)), full((1, NODE_DIM)),
                  full((NODE_DIM, HIDDEN)), full((1, HIDDEN)),
                  full((1, NODE_DIM)), full((1, NODE_DIM)),
                  full((1, EDGE_DIM)), full((1, EDGE_DIM))],
        out_specs=[row(NODE_DIM), row(HIDDEN), row(EDGE_DIM)],
        out_shape=[jax.ShapeDtypeStruct((N_PAD, NODE_DIM), jnp.float32),
                   jax.ShapeDtypeStruct((N_PAD, HIDDEN), jnp.float32),
                   jax.ShapeDtypeStruct((N_PAD, EDGE_DIM), jnp.float32)],
    )(x_scalar, x_spherical, W1, b1.reshape(1, -1), W2, b2.reshape(1, -1),
      ln_g.reshape(1, -1), ln_b.reshape(1, -1), col_w, col_b)


def _edge_body(gsc_ref, gsp_ref, rbf_ref, rsh_ref, rbfw_ref, w0_ref, w1_ref,
               w2_ref, msc_ref, ms0_ref, ms1_ref, ms2_ref):
    f32 = jnp.float32
    bf = jnp.bfloat16
    fw = jnp.dot(rbf_ref[...].astype(bf), rbfw_ref[...].astype(bf),
                 preferred_element_type=f32)
    fo = gsc_ref[...][:, :HIDDEN] * fw
    gate = fo[:, :NUM_IRREPS]
    msc_ref[...] = fo[:, NUM_IRREPS:]

    gsp = gsp_ref[...]
    g0 = gate[:, :128]
    g1 = gate[:, 128:192]
    g2 = gate[:, 192:224]
    x0 = (gsp[:, :128] * g0).astype(bf)
    x1 = [(gsp[:, 128 + 64 * i:128 + 64 * (i + 1)] * g1).astype(bf)
          for i in range(3)]
    x2 = [(gsp[:, 320 + 32 * i:320 + 32 * (i + 1)] * g2).astype(bf)
          for i in range(5)]
    W0 = w0_ref[...].astype(bf)
    W1 = w1_ref[...].astype(bf)
    W2 = w2_ref[...].astype(bf)
    Y = {0: [jnp.dot(x0, W0, preferred_element_type=f32)],
         1: [jnp.dot(x, W1, preferred_element_type=f32) for x in x1],
         2: [jnp.dot(x, W2, preferred_element_type=f32) for x in x2]}

    rsh = rsh_ref[...]
    acc = {0: [None], 1: [None] * 3, 2: [None] * 5}
    for (l1, l2, io, yoff, mo, terms) in _COMBO:
        rbase = _RBASE[l2]
        for (i, k), jl in sorted(terms.items()):
            kv = None
            for (j, c) in jl:
                t = c * rsh[:, rbase + j:rbase + j + 1]
                kv = t if kv is None else kv + t
            contrib = kv * Y[l1][i][:, yoff:yoff + mo]
            acc[io][k] = contrib if acc[io][k] is None else acc[io][k] + contrib
    sph = jnp.concatenate(acc[0] + acc[1] + acc[2], axis=1)
    ms0_ref[...] = sph[:, 0:160]
    ms1_ref[...] = sph[:, 160:320]
    ms2_ref[...] = sph[:, 320:480]


def _edge_stage(gsc, gsp, rbf, rsh, rbf_w, W0, W1, W2):
    nblk = N_EDGES // EDGE_BLK
    full = lambda shape: pl.BlockSpec(shape, lambda i: (0, 0))
    row = lambda n: pl.BlockSpec((EDGE_BLK, n), lambda i: (i, 0))
    return pl.pallas_call(
        _edge_body,
        grid=(nblk,),
        in_specs=[row(HID_P), row(SPH_P), row(NUM_BASIS), row(SPH_DIM),
                  full((NUM_BASIS, HIDDEN)), full((128, 224)),
                  full((64, 384)), full((32, 352))],
        out_specs=[row(NODE_DIM), row(160), row(160), row(160)],
        out_shape=[jax.ShapeDtypeStruct((N_EDGES, NODE_DIM), jnp.float32),
                   jax.ShapeDtypeStruct((N_EDGES, 160), jnp.float32),
                   jax.ShapeDtypeStruct((N_EDGES, 160), jnp.float32),
                   jax.ShapeDtypeStruct((N_EDGES, 160), jnp.float32)],
    )(gsc, gsp, rbf, rsh, rbf_w, W0, W1, W2)


_SC_MESH = plsc.VectorSubcoreMesh(core_axis_name="c", subcore_axis_name="s",
                                  num_cores=2, num_subcores=16)
_NW = 32
_GC = 128
_NCHUNK = N_EDGES // _GC
HID_P = 384
SPH_P = 512


def _sc_gather(src_idx, scalar_tbl, sph_tbl):
    base_chunks = _NCHUNK // _NW
    rem = _NCHUNK % _NW

    @functools.partial(
        pl.kernel,
        out_type=[jax.ShapeDtypeStruct((N_EDGES, HID_P), jnp.float32),
                  jax.ShapeDtypeStruct((N_EDGES, SPH_P), jnp.float32)],
        mesh=_SC_MESH,
    )
    def gk(idx_hbm, t1_hbm, t2_hbm, o1_hbm, o2_hbm):
        wid = jax.lax.axis_index("s") * 2 + jax.lax.axis_index("c")
        nmine = base_chunks + jnp.where(wid < rem, 1, 0)

        def phase(tbl, out, width):
            def body(idx_v, rows_v, sem):
                def step(i, _):
                    e0 = (wid + i * _NW) * _GC
                    pltpu.sync_copy(idx_hbm.at[pl.ds(e0, _GC)], idx_v)
                    pltpu.async_copy(tbl.at[idx_v], rows_v, sem).wait()
                    pltpu.sync_copy(rows_v, out.at[pl.ds(e0, _GC)])
                    return 0
                jax.lax.fori_loop(0, nmine, step, 0)
            pl.run_scoped(body,
                          pltpu.VMEM((_GC,), jnp.int32),
                          pltpu.VMEM((_GC, width), jnp.float32),
                          pltpu.SemaphoreType.DMA)

        phase(t1_hbm, o1_hbm, HID_P)
        phase(t2_hbm, o2_hbm, SPH_P)

    return gk(src_idx, scalar_tbl, sph_tbl)


def _to_imajor(sph):
    n = sph.shape[0]
    l1 = sph[:, 128:320].reshape(n, 64, 3).transpose(0, 2, 1).reshape(n, 192)
    l2 = sph[:, 320:480].reshape(n, 32, 5).transpose(0, 2, 1).reshape(n, 160)
    return jnp.concatenate([sph[:, :128], l1, l2], axis=1)


def _from_imajor(sph):
    n = sph.shape[0]
    l1 = sph[:, 128:320].reshape(n, 3, 64).transpose(0, 2, 1).reshape(n, 192)
    l2 = sph[:, 320:480].reshape(n, 5, 32).transpose(0, 2, 1).reshape(n, 160)
    return jnp.concatenate([sph[:, :128], l1, l2], axis=1)


def kernel(x_scalar, x_spherical, rbf, rsh, W1, b1, W2, b2, rbf_w, ln_g, ln_b,
           o3_w, o3_b, tp_w, edge_index):
    col_w = jnp.concatenate([
        o3_w[:128],
        jnp.repeat(o3_w[128:192], 3),
        jnp.repeat(o3_w[192:224], 5)]).reshape(1, EDGE_DIM)
    col_b = jnp.concatenate(
        [o3_b, jnp.zeros((EDGE_DIM - 128,), jnp.float32)]).reshape(1, EDGE_DIM)

    xs = jnp.pad(x_scalar, ((0, N_PAD - N_NODES), (0, 0)))
    xsp = jnp.pad(x_spherical, ((0, N_PAD - N_NODES), (0, 0)))
    scalar_in, scalar_out, sph_in = _node_stage(
        xs, xsp, W1, b1, W2, b2, ln_g, ln_b, col_w, col_b)
    sph_in_im = _to_imajor(sph_in)

    W0c, W1c, W2c = _prep_tp_weights(tp_w)

    src = edge_index[1]
    dst = edge_index[0]
    sc_tbl = jnp.pad(scalar_out, ((0, 0), (0, HID_P - HIDDEN)))
    sp_tbl = jnp.pad(sph_in_im, ((0, 0), (0, SPH_P - EDGE_DIM)))
    gsc, gsp = _sc_gather(src, sc_tbl, sp_tbl)

    msc, ms0, ms1, ms2 = _edge_stage(gsc, gsp, rbf, rsh, rbf_w, W0c, W1c, W2c)
    msph = jnp.concatenate([ms0, ms1, ms2], axis=1)

    new_scalar = scalar_in[:N_NODES].at[dst].add(msc)
    new_sph_im = sph_in_im[:N_NODES].at[dst].add(msph)
    return new_scalar, _from_imajor(new_sph_im)


# speedup vs baseline: 3.1846x; 2.0835x over previous
import functools
from math import factorial

import jax
import jax.numpy as jnp
import numpy as np
from jax.experimental import pallas as pl
from jax.experimental.pallas import tpu as pltpu
from jax.experimental.pallas import tpu_sc as plsc

NODE_DIM = 128
NUM_BASIS = 20
IRREPS = [(128, 0), (64, 1), (32, 2)]
SPH = [(1, 0), (1, 1), (1, 2)]
NUM_IRREPS = sum(m for m, _ in IRREPS)
EDGE_DIM = sum(m * (2 * l + 1) for m, l in IRREPS)
SPH_DIM = sum(m * (2 * l + 1) for m, l in SPH)
HIDDEN = NODE_DIM + NUM_IRREPS
N_NODES = 10000
N_EDGES = 160000
N_PAD = 10240

NODE_BLK = 128
EDGE_BLK = 256


def _su2_cg(j1, j2, j3, m1, m2, m3):
    if m3 != m1 + m2:
        return 0.0
    vmin = int(max(-j1 + j2 + m3, -j1 + m1, 0))
    vmax = int(min(j2 + j3 + m1, j3 - j1 + j2, j3 + m3))

    def f(n):
        return float(factorial(round(n)))

    C = ((2 * j3 + 1) * f(j3 + j1 - j2) * f(j3 - j1 + j2) * f(j1 + j2 - j3) / f(j1 + j2 + j3 + 1)
         * f(j3 + m3) * f(j3 - m3) / (f(j1 - m1) * f(j1 + m1) * f(j2 - m2) * f(j2 + m2))) ** 0.5
    S = 0.0
    for v in range(vmin, vmax + 1):
        S += (-1.0) ** (v + j2 + m2) / f(v) * f(j2 + j3 + m1 - v) * f(j1 - m1 + v) / (
            f(j3 - j1 + j2 - v) * f(j3 + m3 - v) * f(v + j1 - j2 - m3))
    return C * S


def _su2_cg_tensor(l1, l2, l3):
    C = np.zeros((2 * l1 + 1, 2 * l2 + 1, 2 * l3 + 1))
    for m1 in range(-l1, l1 + 1):
        for m2 in range(-l2, l2 + 1):
            m3 = m1 + m2
            if abs(m3) <= l3:
                C[m1 + l1, m2 + l2, m3 + l3] = _su2_cg(l1, l2, l3, m1, m2, m3)
    return C


def _q_mat(l):
    q = np.zeros((2 * l + 1, 2 * l + 1), dtype=complex)
    for m in range(-l, 0):
        q[l + m, l + abs(m)] = 1 / 2 ** 0.5
        q[l + m, l - abs(m)] = -1j / 2 ** 0.5
    q[l, l] = 1.0
    for m in range(1, l + 1):
        q[l + m, l + abs(m)] = (-1) ** m / 2 ** 0.5
        q[l + m, l - abs(m)] = 1j * (-1) ** m / 2 ** 0.5
    return (-1j) ** l * q


def _w3j(l1, l2, l3):
    C = _su2_cg_tensor(l1, l2, l3).astype(complex)
    Q1, Q2, Q3 = _q_mat(l1), _q_mat(l2), _q_mat(l3)
    C = np.einsum('ij,kl,mn,ikm->jln', Q1, Q2, np.conj(Q3), C)
    re, im = np.real(C), np.imag(C)
    C = re if np.abs(re).sum() >= np.abs(im).sum() else im
    n = np.linalg.norm(C)
    return C / n if n > 0 else C


_PATHS = []
for _i1, (_m1, _l1) in enumerate(IRREPS):
    for _i2, (_m2, _l2) in enumerate(SPH):
        for _io, (_mo, _l3) in enumerate(IRREPS):
            if abs(_l1 - _l2) <= _l3 <= _l1 + _l2:
                _PATHS.append((_i1, _i2, _io))
_FAN_IN = [0] * len(IRREPS)
for (_i1, _i2, _io) in _PATHS:
    _FAN_IN[_io] += IRREPS[_i1][0] * SPH[_i2][0]
_W3J = {}
for (_i1, _i2, _io) in _PATHS:
    _k = (IRREPS[_i1][1], SPH[_i2][1], IRREPS[_io][1])
    if _k not in _W3J:
        _W3J[_k] = _w3j(*_k)

_YW = {0: 0, 1: 0, 2: 0}
_COMBO = []
for (_i1, _i2, _io) in _PATHS:
    _m1, _l1 = IRREPS[_i1]
    _, _l2 = SPH[_i2]
    _mo, _l3 = IRREPS[_io]
    _cg = _W3J[(_l1, _l2, _l3)]
    _terms = {}
    for _i in range(2 * _l1 + 1):
        for _j in range(2 * _l2 + 1):
            for _kk in range(2 * _l3 + 1):
                _c = _cg[_i, _j, _kk]
                if abs(_c) > 1e-12:
                    _terms.setdefault((_i, _kk), []).append((_j, float(_c)))
    _COMBO.append((_l1, _l2, _io, _YW[_l1], _mo, _terms))
    _YW[_l1] += _mo

_RBASE = {0: 0, 1: 1, 2: 4}


def _prep_tp_weights(tp_w):
    groups = {0: [], 1: [], 2: []}
    off = 0
    for (i1, i2, io) in _PATHS:
        m1, l1 = IRREPS[i1]
        mo, l3 = IRREPS[io]
        w = tp_w[off:off + m1 * mo].reshape(m1, mo)
        off += m1 * mo
        alpha = (2 * l3 + 1) ** 0.5 / _FAN_IN[io] ** 0.5
        groups[l1].append(w * alpha)
    return (jnp.concatenate(groups[0], axis=1),
            jnp.concatenate(groups[1], axis=1),
            jnp.concatenate(groups[2], axis=1))


def _node_body(xs_ref, xsp_ref, w1_ref, b1_ref, w2_ref, b2_ref, lng_ref,
               lnb_ref, colw_ref, colb_ref, sin_ref, sout_ref, sphn_ref):
    f32 = jnp.float32
    bf = jnp.bfloat16
    x = xs_ref[...]
    mu = jnp.mean(x, axis=1, keepdims=True)
    xc = x - mu
    var = jnp.mean(xc * xc, axis=1, keepdims=True)
    sin = xc / jnp.sqrt(var + 1e-5) * lng_ref[...] + lnb_ref[...]
    sin_ref[...] = sin
    h = jnp.dot(sin.astype(bf), w1_ref[...].astype(bf),
                preferred_element_type=f32) + b1_ref[...]
    h = h * jax.nn.sigmoid(h)
    sout_ref[...] = jnp.dot(h.astype(bf), w2_ref[...].astype(bf),
                            preferred_element_type=f32) + b2_ref[...]

    sp = xsp_ref[...]
    v = sp[:, :128]
    mu0 = jnp.mean(v, axis=1, keepdims=True)
    v = v - mu0
    o0 = v / jnp.sqrt(jnp.mean(v * v, axis=1, keepdims=True) + 1e-5)
    blk1 = sp[:, 128:320]
    n1 = jnp.sum(blk1 * blk1, axis=1, keepdims=True) * (1.0 / 64.0)
    o1 = blk1 / jnp.sqrt(n1 + 1e-5)
    blk2 = sp[:, 320:480]
    n2 = jnp.sum(blk2 * blk2, axis=1, keepdims=True) * (1.0 / 32.0)
    o2 = blk2 / jnp.sqrt(n2 + 1e-5)
    out = jnp.concatenate([o0, o1, o2], axis=1) * colw_ref[...] + colb_ref[...]
    sphn_ref[...] = out


def _node_stage(x_scalar, x_spherical, W1, b1, W2, b2, ln_g, ln_b, col_w, col_b):
    nblk = N_PAD // NODE_BLK
    full = lambda shape: pl.BlockSpec(shape, lambda i: (0, 0))
    row = lambda n: pl.BlockSpec((NODE_BLK, n), lambda i: (i, 0))
    return pl.pallas_call(
        _node_body,
        grid=(nblk,),
        in_specs=[row(NODE_DIM), row(EDGE_DIM),
                  full((NODE_DIM, NODE_DIM)), full((1, NODE_DIM)),
                  full((NODE_DIM, HIDDEN)), full((1, HIDDEN)),
                  full((1, NODE_DIM)), full((1, NODE_DIM)),
                  full((1, EDGE_DIM)), full((1, EDGE_DIM))],
        out_specs=[row(NODE_DIM), row(HIDDEN), row(EDGE_DIM)],
        out_shape=[jax.ShapeDtypeStruct((N_PAD, NODE_DIM), jnp.float32),
                   jax.ShapeDtypeStruct((N_PAD, HIDDEN), jnp.float32),
                   jax.ShapeDtypeStruct((N_PAD, EDGE_DIM), jnp.float32)],
    )(x_scalar, x_spherical, W1, b1.reshape(1, -1), W2, b2.reshape(1, -1),
      ln_g.reshape(1, -1), ln_b.reshape(1, -1), col_w, col_b)


def _edge_body(gsc_ref, gsp_ref, rbf_ref, rsht_ref, rbfw_ref, w0_ref, w1_ref,
               w2_ref, msc_ref, ms0_ref, ms1_ref, ms2_ref):
    f32 = jnp.float32
    bf = jnp.bfloat16
    fw = jnp.dot(rbf_ref[...].astype(bf), rbfw_ref[...].astype(bf),
                 preferred_element_type=f32)
    fo = gsc_ref[...][:, :HIDDEN] * fw
    msc_ref[...] = fo[:, NUM_IRREPS:]
    gt = fo[:, :NUM_IRREPS].T
    gsp = gsp_ref[...][:, :EDGE_DIM].T

    g0 = gt[:128, :]
    g1 = gt[128:192, :]
    g2 = gt[192:224, :]
    x0 = (gsp[:128, :] * g0).astype(bf)
    x1 = [(gsp[128 + 64 * i:128 + 64 * (i + 1), :] * g1).astype(bf)
          for i in range(3)]
    x2 = [(gsp[320 + 32 * i:320 + 32 * (i + 1), :] * g2).astype(bf)
          for i in range(5)]
    W0 = w0_ref[...].astype(bf)
    W1 = w1_ref[...].astype(bf)
    W2 = w2_ref[...].astype(bf)
    Y = {0: [jnp.dot(W0, x0, preferred_element_type=f32)],
         1: [jnp.dot(W1, x, preferred_element_type=f32) for x in x1],
         2: [jnp.dot(W2, x, preferred_element_type=f32) for x in x2]}

    rsh = rsht_ref[...]
    acc = {0: [None], 1: [None] * 3, 2: [None] * 5}
    for (l1, l2, io, yoff, mo, terms) in _COMBO:
        rbase = _RBASE[l2]
        for (i, k), jl in sorted(terms.items()):
            kv = None
            for (j, c) in jl:
                t = c * rsh[rbase + j:rbase + j + 1, :]
                kv = t if kv is None else kv + t
            contrib = kv * Y[l1][i][yoff:yoff + mo, :]
            acc[io][k] = contrib if acc[io][k] is None else acc[io][k] + contrib
    sph = jnp.concatenate(acc[0] + acc[1] + acc[2], axis=0)
    ms0_ref[...] = sph[0:160, :].T
    ms1_ref[...] = sph[160:320, :].T
    ms2_ref[...] = sph[320:480, :].T


def _edge_stage(gsc, gsp, rbf, rsh_t, rbf_w, W0t, W1t, W2t):
    nblk = N_EDGES // EDGE_BLK
    full = lambda shape: pl.BlockSpec(shape, lambda i: (0, 0))
    row = lambda n: pl.BlockSpec((EDGE_BLK, n), lambda i: (i, 0))
    col = lambda n: pl.BlockSpec((n, EDGE_BLK), lambda i: (0, i))
    return pl.pallas_call(
        _edge_body,
        grid=(nblk,),
        in_specs=[row(HID_P), row(SPH_P), row(NUM_BASIS), col(16),
                  full((NUM_BASIS, HIDDEN)), full((224, 128)),
                  full((384, 64)), full((352, 32))],
        out_specs=[row(NODE_DIM), row(160), row(160), row(160)],
        out_shape=[jax.ShapeDtypeStruct((N_EDGES, NODE_DIM), jnp.float32),
                   jax.ShapeDtypeStruct((N_EDGES, 160), jnp.float32),
                   jax.ShapeDtypeStruct((N_EDGES, 160), jnp.float32),
                   jax.ShapeDtypeStruct((N_EDGES, 160), jnp.float32)],
    )(gsc, gsp, rbf, rsh_t, rbf_w, W0t, W1t, W2t)


_SC_MESH = plsc.VectorSubcoreMesh(core_axis_name="c", subcore_axis_name="s",
                                  num_cores=2, num_subcores=16)
_NW = 32
_GC = 128
_NCHUNK = N_EDGES // _GC
HID_P = 384
SPH_P = 512


def _sc_gather(src_idx, scalar_tbl, sph_tbl):
    base_chunks = _NCHUNK // _NW
    rem = _NCHUNK % _NW

    @functools.partial(
        pl.kernel,
        out_type=[jax.ShapeDtypeStruct((N_EDGES, HID_P), jnp.float32),
                  jax.ShapeDtypeStruct((N_EDGES, SPH_P), jnp.float32)],
        mesh=_SC_MESH,
    )
    def gk(idx_hbm, t1_hbm, t2_hbm, o1_hbm, o2_hbm):
        wid = jax.lax.axis_index("s") * 2 + jax.lax.axis_index("c")
        nmine = base_chunks + jnp.where(wid < rem, 1, 0)

        def phase(tbl, out, width):
            def body(idx_v, rows_v, sem):
                def step(i, _):
                    e0 = (wid + i * _NW) * _GC
                    pltpu.sync_copy(idx_hbm.at[pl.ds(e0, _GC)], idx_v)
                    pltpu.async_copy(tbl.at[idx_v], rows_v, sem).wait()
                    pltpu.sync_copy(rows_v, out.at[pl.ds(e0, _GC)])
                    return 0
                jax.lax.fori_loop(0, nmine, step, 0)
            pl.run_scoped(body,
                          pltpu.VMEM((_GC,), jnp.int32),
                          pltpu.VMEM((_GC, width), jnp.float32),
                          pltpu.SemaphoreType.DMA)

        phase(t1_hbm, o1_hbm, HID_P)
        phase(t2_hbm, o2_hbm, SPH_P)

    return gk(src_idx, scalar_tbl, sph_tbl)


def _to_imajor(sph):
    n = sph.shape[0]
    l1 = sph[:, 128:320].reshape(n, 64, 3).transpose(0, 2, 1).reshape(n, 192)
    l2 = sph[:, 320:480].reshape(n, 32, 5).transpose(0, 2, 1).reshape(n, 160)
    return jnp.concatenate([sph[:, :128], l1, l2], axis=1)


def _from_imajor(sph):
    n = sph.shape[0]
    l1 = sph[:, 128:320].reshape(n, 3, 64).transpose(0, 2, 1).reshape(n, 192)
    l2 = sph[:, 320:480].reshape(n, 5, 32).transpose(0, 2, 1).reshape(n, 160)
    return jnp.concatenate([sph[:, :128], l1, l2], axis=1)


def kernel(x_scalar, x_spherical, rbf, rsh, W1, b1, W2, b2, rbf_w, ln_g, ln_b,
           o3_w, o3_b, tp_w, edge_index):
    col_w = jnp.concatenate([
        o3_w[:128],
        jnp.repeat(o3_w[128:192], 3),
        jnp.repeat(o3_w[192:224], 5)]).reshape(1, EDGE_DIM)
    col_b = jnp.concatenate(
        [o3_b, jnp.zeros((EDGE_DIM - 128,), jnp.float32)]).reshape(1, EDGE_DIM)

    xs = jnp.pad(x_scalar, ((0, N_PAD - N_NODES), (0, 0)))
    xsp = jnp.pad(x_spherical, ((0, N_PAD - N_NODES), (0, 0)))
    scalar_in, scalar_out, sph_in = _node_stage(
        xs, xsp, W1, b1, W2, b2, ln_g, ln_b, col_w, col_b)
    sph_in_im = _to_imajor(sph_in)

    W0c, W1c, W2c = _prep_tp_weights(tp_w)

    src = edge_index[1]
    dst = edge_index[0]
    sc_tbl = jnp.pad(scalar_out, ((0, 0), (0, HID_P - HIDDEN)))
    sp_tbl = jnp.pad(sph_in_im, ((0, 0), (0, SPH_P - EDGE_DIM)))
    gsc, gsp = _sc_gather(src, sc_tbl, sp_tbl)

    rsh_t = jnp.pad(rsh.T, ((0, 16 - SPH_DIM), (0, 0)))
    msc, ms0, ms1, ms2 = _edge_stage(gsc, gsp, rbf, rsh_t, rbf_w,
                                     W0c.T, W1c.T, W2c.T)
    msph = jnp.concatenate([ms0, ms1, ms2], axis=1)

    new_scalar = scalar_in[:N_NODES].at[dst].add(msc)
    new_sph_im = sph_in_im[:N_NODES].at[dst].add(msph)
    return new_scalar, _from_imajor(new_sph_im)


# speedup vs baseline: 4.7395x; 1.4882x over previous
import functools
from math import factorial

import jax
import jax.numpy as jnp
import numpy as np
from jax.experimental import pallas as pl
from jax.experimental.pallas import tpu as pltpu
from jax.experimental.pallas import tpu_sc as plsc

NODE_DIM = 128
NUM_BASIS = 20
IRREPS = [(128, 0), (64, 1), (32, 2)]
SPH = [(1, 0), (1, 1), (1, 2)]
NUM_IRREPS = sum(m for m, _ in IRREPS)
EDGE_DIM = sum(m * (2 * l + 1) for m, l in IRREPS)
SPH_DIM = sum(m * (2 * l + 1) for m, l in SPH)
HIDDEN = NODE_DIM + NUM_IRREPS
N_NODES = 10000
N_EDGES = 160000
N_PAD = 10240

NODE_BLK = 128
EDGE_BLK = 256


def _su2_cg(j1, j2, j3, m1, m2, m3):
    if m3 != m1 + m2:
        return 0.0
    vmin = int(max(-j1 + j2 + m3, -j1 + m1, 0))
    vmax = int(min(j2 + j3 + m1, j3 - j1 + j2, j3 + m3))

    def f(n):
        return float(factorial(round(n)))

    C = ((2 * j3 + 1) * f(j3 + j1 - j2) * f(j3 - j1 + j2) * f(j1 + j2 - j3) / f(j1 + j2 + j3 + 1)
         * f(j3 + m3) * f(j3 - m3) / (f(j1 - m1) * f(j1 + m1) * f(j2 - m2) * f(j2 + m2))) ** 0.5
    S = 0.0
    for v in range(vmin, vmax + 1):
        S += (-1.0) ** (v + j2 + m2) / f(v) * f(j2 + j3 + m1 - v) * f(j1 - m1 + v) / (
            f(j3 - j1 + j2 - v) * f(j3 + m3 - v) * f(v + j1 - j2 - m3))
    return C * S


def _su2_cg_tensor(l1, l2, l3):
    C = np.zeros((2 * l1 + 1, 2 * l2 + 1, 2 * l3 + 1))
    for m1 in range(-l1, l1 + 1):
        for m2 in range(-l2, l2 + 1):
            m3 = m1 + m2
            if abs(m3) <= l3:
                C[m1 + l1, m2 + l2, m3 + l3] = _su2_cg(l1, l2, l3, m1, m2, m3)
    return C


def _q_mat(l):
    q = np.zeros((2 * l + 1, 2 * l + 1), dtype=complex)
    for m in range(-l, 0):
        q[l + m, l + abs(m)] = 1 / 2 ** 0.5
        q[l + m, l - abs(m)] = -1j / 2 ** 0.5
    q[l, l] = 1.0
    for m in range(1, l + 1):
        q[l + m, l + abs(m)] = (-1) ** m / 2 ** 0.5
        q[l + m, l - abs(m)] = 1j * (-1) ** m / 2 ** 0.5
    return (-1j) ** l * q


def _w3j(l1, l2, l3):
    C = _su2_cg_tensor(l1, l2, l3).astype(complex)
    Q1, Q2, Q3 = _q_mat(l1), _q_mat(l2), _q_mat(l3)
    C = np.einsum('ij,kl,mn,ikm->jln', Q1, Q2, np.conj(Q3), C)
    re, im = np.real(C), np.imag(C)
    C = re if np.abs(re).sum() >= np.abs(im).sum() else im
    n = np.linalg.norm(C)
    return C / n if n > 0 else C


_PATHS = []
for _i1, (_m1, _l1) in enumerate(IRREPS):
    for _i2, (_m2, _l2) in enumerate(SPH):
        for _io, (_mo, _l3) in enumerate(IRREPS):
            if abs(_l1 - _l2) <= _l3 <= _l1 + _l2:
                _PATHS.append((_i1, _i2, _io))
_FAN_IN = [0] * len(IRREPS)
for (_i1, _i2, _io) in _PATHS:
    _FAN_IN[_io] += IRREPS[_i1][0] * SPH[_i2][0]
_W3J = {}
for (_i1, _i2, _io) in _PATHS:
    _k = (IRREPS[_i1][1], SPH[_i2][1], IRREPS[_io][1])
    if _k not in _W3J:
        _W3J[_k] = _w3j(*_k)

_YW = {0: 0, 1: 0, 2: 0}
_COMBO = []
for (_i1, _i2, _io) in _PATHS:
    _m1, _l1 = IRREPS[_i1]
    _, _l2 = SPH[_i2]
    _mo, _l3 = IRREPS[_io]
    _cg = _W3J[(_l1, _l2, _l3)]
    _terms = {}
    for _i in range(2 * _l1 + 1):
        for _j in range(2 * _l2 + 1):
            for _kk in range(2 * _l3 + 1):
                _c = _cg[_i, _j, _kk]
                if abs(_c) > 1e-12:
                    _terms.setdefault((_i, _kk), []).append((_j, float(_c)))
    _COMBO.append((_l1, _l2, _io, _YW[_l1], _mo, _terms))
    _YW[_l1] += _mo

_RBASE = {0: 0, 1: 1, 2: 4}


def _prep_tp_weights(tp_w):
    groups = {0: [], 1: [], 2: []}
    off = 0
    for (i1, i2, io) in _PATHS:
        m1, l1 = IRREPS[i1]
        mo, l3 = IRREPS[io]
        w = tp_w[off:off + m1 * mo].reshape(m1, mo)
        off += m1 * mo
        alpha = (2 * l3 + 1) ** 0.5 / _FAN_IN[io] ** 0.5
        groups[l1].append(w * alpha)
    return (jnp.concatenate(groups[0], axis=1),
            jnp.concatenate(groups[1], axis=1),
            jnp.concatenate(groups[2], axis=1))


def _node_body(xs_ref, xsp_ref, w1_ref, b1_ref, w2_ref, b2_ref, lng_ref,
               lnb_ref, colw_ref, colb_ref, sin_ref, sout_ref, sphn_ref):
    f32 = jnp.float32
    bf = jnp.bfloat16
    x = xs_ref[...]
    mu = jnp.mean(x, axis=1, keepdims=True)
    xc = x - mu
    var = jnp.mean(xc * xc, axis=1, keepdims=True)
    sin = xc / jnp.sqrt(var + 1e-5) * lng_ref[...] + lnb_ref[...]
    sin_ref[...] = sin
    h = jnp.dot(sin.astype(bf), w1_ref[...].astype(bf),
                preferred_element_type=f32) + b1_ref[...]
    h = h * jax.nn.sigmoid(h)
    sout_ref[...] = jnp.dot(h.astype(bf), w2_ref[...].astype(bf),
                            preferred_element_type=f32) + b2_ref[...]

    sp = xsp_ref[...]
    v = sp[:, :128]
    mu0 = jnp.mean(v, axis=1, keepdims=True)
    v = v - mu0
    o0 = v / jnp.sqrt(jnp.mean(v * v, axis=1, keepdims=True) + 1e-5)
    blk1 = sp[:, 128:320]
    n1 = jnp.sum(blk1 * blk1, axis=1, keepdims=True) * (1.0 / 64.0)
    o1 = blk1 / jnp.sqrt(n1 + 1e-5)
    blk2 = sp[:, 320:480]
    n2 = jnp.sum(blk2 * blk2, axis=1, keepdims=True) * (1.0 / 32.0)
    o2 = blk2 / jnp.sqrt(n2 + 1e-5)
    out = jnp.concatenate([o0, o1, o2], axis=1) * colw_ref[...] + colb_ref[...]
    sphn_ref[...] = out


def _node_stage(x_scalar, x_spherical, W1, b1, W2, b2, ln_g, ln_b, col_w, col_b):
    nblk = N_PAD // NODE_BLK
    full = lambda shape: pl.BlockSpec(shape, lambda i: (0, 0))
    row = lambda n: pl.BlockSpec((NODE_BLK, n), lambda i: (i, 0))
    return pl.pallas_call(
        _node_body,
        grid=(nblk,),
        in_specs=[row(NODE_DIM), row(EDGE_DIM),
                  full((NODE_DIM, NODE_DIM)), full((1, NODE_DIM)),
                  full((NODE_DIM, HIDDEN)), full((1, HIDDEN)),
                  full((1, NODE_DIM)), full((1, NODE_DIM)),
                  full((1, EDGE_DIM)), full((1, EDGE_DIM))],
        out_specs=[row(NODE_DIM), row(HIDDEN), row(EDGE_DIM)],
        out_shape=[jax.ShapeDtypeStruct((N_PAD, NODE_DIM), jnp.float32),
                   jax.ShapeDtypeStruct((N_PAD, HIDDEN), jnp.float32),
                   jax.ShapeDtypeStruct((N_PAD, EDGE_DIM), jnp.float32)],
    )(x_scalar, x_spherical, W1, b1.reshape(1, -1), W2, b2.reshape(1, -1),
      ln_g.reshape(1, -1), ln_b.reshape(1, -1), col_w, col_b)


def _edge_body(gsc_ref, gsp_ref, rbf_ref, rsht_ref, rbfw_ref, w0_ref, w1_ref,
               w2_ref, msc_ref, ms0_ref, ms1_ref, ms2_ref, ms3_ref):
    f32 = jnp.float32
    bf = jnp.bfloat16
    fw = jnp.dot(rbf_ref[...].astype(bf), rbfw_ref[...].astype(bf),
                 preferred_element_type=f32)
    fo = gsc_ref[...][:, :HIDDEN] * fw
    msc_ref[...] = fo[:, NUM_IRREPS:]
    gt = fo[:, :NUM_IRREPS].T
    gsp = gsp_ref[...][:, :EDGE_DIM].T

    g0 = gt[:128, :]
    g1 = gt[128:192, :]
    g2 = gt[192:224, :]
    x0 = (gsp[:128, :] * g0).astype(bf)
    x1 = [(gsp[128 + 64 * i:128 + 64 * (i + 1), :] * g1).astype(bf)
          for i in range(3)]
    x2 = [(gsp[320 + 32 * i:320 + 32 * (i + 1), :] * g2).astype(bf)
          for i in range(5)]
    W0 = w0_ref[...].astype(bf)
    W1 = w1_ref[...].astype(bf)
    W2 = w2_ref[...].astype(bf)
    Y = {0: [jnp.dot(W0, x0, preferred_element_type=f32)],
         1: [jnp.dot(W1, x, preferred_element_type=f32) for x in x1],
         2: [jnp.dot(W2, x, preferred_element_type=f32) for x in x2]}

    rsh = rsht_ref[...]
    acc = {0: [None], 1: [None] * 3, 2: [None] * 5}
    for (l1, l2, io, yoff, mo, terms) in _COMBO:
        rbase = _RBASE[l2]
        for (i, k), jl in sorted(terms.items()):
            kv = None
            for (j, c) in jl:
                t = c * rsh[rbase + j:rbase + j + 1, :]
                kv = t if kv is None else kv + t
            contrib = kv * Y[l1][i][yoff:yoff + mo, :]
            acc[io][k] = contrib if acc[io][k] is None else acc[io][k] + contrib
    sph = jnp.concatenate(acc[0] + acc[1] + acc[2], axis=0)
    ms0_ref[...] = sph[0:128, :].T
    ms1_ref[...] = sph[128:256, :].T
    ms2_ref[...] = sph[256:384, :].T
    ms3_ref[...] = jnp.concatenate(
        [sph[384:480, :], jnp.zeros_like(sph[:32, :])], axis=0).T


def _edge_stage(gsc, gsp, rbf, rsh_t, rbf_w, W0t, W1t, W2t):
    nblk = N_EDGES // EDGE_BLK
    full = lambda shape: pl.BlockSpec(shape, lambda i: (0, 0))
    row = lambda n: pl.BlockSpec((EDGE_BLK, n), lambda i: (i, 0))
    col = lambda n: pl.BlockSpec((n, EDGE_BLK), lambda i: (0, i))
    return pl.pallas_call(
        _edge_body,
        grid=(nblk,),
        in_specs=[row(HID_P), row(SPH_P), row(NUM_BASIS), col(16),
                  full((NUM_BASIS, HIDDEN)), full((224, 128)),
                  full((384, 64)), full((352, 32))],
        out_specs=[row(128)] * 5,
        out_shape=[jax.ShapeDtypeStruct((N_EDGES, 128), jnp.float32)] * 5,
    )(gsc, gsp, rbf, rsh_t, rbf_w, W0t, W1t, W2t)


_SC_MESH = plsc.VectorSubcoreMesh(core_axis_name="c", subcore_axis_name="s",
                                  num_cores=2, num_subcores=16)
_NW = 32
_GC = 128
_NCHUNK = N_EDGES // _GC
HID_P = 384
SPH_P = 512


def _sc_gather(src_idx, scalar_tbl, sph_tbl):
    base_chunks = _NCHUNK // _NW
    rem = _NCHUNK % _NW

    @functools.partial(
        pl.kernel,
        out_type=[jax.ShapeDtypeStruct((N_EDGES, HID_P), jnp.float32),
                  jax.ShapeDtypeStruct((N_EDGES, SPH_P), jnp.float32)],
        mesh=_SC_MESH,
    )
    def gk(idx_hbm, t1_hbm, t2_hbm, o1_hbm, o2_hbm):
        wid = jax.lax.axis_index("s") * 2 + jax.lax.axis_index("c")
        nmine = base_chunks + jnp.where(wid < rem, 1, 0)

        def phase(tbl, out, width):
            def body(idx_v, rows_v, sem):
                def step(i, _):
                    e0 = (wid + i * _NW) * _GC
                    pltpu.sync_copy(idx_hbm.at[pl.ds(e0, _GC)], idx_v)
                    pltpu.async_copy(tbl.at[idx_v], rows_v, sem).wait()
                    pltpu.sync_copy(rows_v, out.at[pl.ds(e0, _GC)])
                    return 0
                jax.lax.fori_loop(0, nmine, step, 0)
            pl.run_scoped(body,
                          pltpu.VMEM((_GC,), jnp.int32),
                          pltpu.VMEM((_GC, width), jnp.float32),
                          pltpu.SemaphoreType.DMA)

        phase(t1_hbm, o1_hbm, HID_P)
        phase(t2_hbm, o2_hbm, SPH_P)

    return gk(src_idx, scalar_tbl, sph_tbl)


_SCHUNK = 128
_SNCHUNK = N_EDGES // _SCHUNK
_ROWS_PER_TILE = N_PAD // 16


def _sc_scatter(dst_idx, msgs, bases):
    base_chunks = _SNCHUNK // 16
    rem = _SNCHUNK % 16

    @functools.partial(
        pl.kernel,
        out_type=[jax.ShapeDtypeStruct((N_PAD, 128), jnp.float32)] * 5,
        mesh=_SC_MESH,
        scratch_types=[pltpu.VMEM_SHARED((N_PAD, 128), jnp.float32)],
    )
    def sk(dst_hbm, m0, m1, m2, m3, m4, b0, b1, b2, b3, b4,
           o0, o1, o2, o3, o4, acc_s):
        cid = jax.lax.axis_index("c")
        sid = jax.lax.axis_index("s")
        r0 = sid * _ROWS_PER_TILE
        nmine = base_chunks + jnp.where(sid < rem, 1, 0)

        def chunk(m, b, o):
            pltpu.sync_copy(b.at[pl.ds(r0, _ROWS_PER_TILE)],
                            acc_s.at[pl.ds(r0, _ROWS_PER_TILE)])
            plsc.subcore_barrier()

            def body(idx_v, m_v, sem):
                def step(i, _):
                    e0 = (sid + i * 16) * _SCHUNK
                    pltpu.sync_copy(dst_hbm.at[pl.ds(e0, _SCHUNK)], idx_v)
                    pltpu.async_copy(m.at[pl.ds(e0, _SCHUNK)], m_v, sem).wait()
                    pltpu.sync_copy(m_v, acc_s.at[idx_v], add=True)
                    return 0
                jax.lax.fori_loop(0, nmine, step, 0)
            pl.run_scoped(body,
                          pltpu.VMEM((_SCHUNK,), jnp.int32),
                          pltpu.VMEM((_SCHUNK, 128), jnp.float32),
                          pltpu.SemaphoreType.DMA)
            plsc.subcore_barrier()
            pltpu.sync_copy(acc_s.at[pl.ds(r0, _ROWS_PER_TILE)],
                            o.at[pl.ds(r0, _ROWS_PER_TILE)])
            plsc.subcore_barrier()

        @pl.when(cid == 0)
        def _():
            chunk(m0, b0, o0)
            chunk(m1, b1, o1)
            chunk(m2, b2, o2)

        @pl.when(cid == 1)
        def _():
            chunk(m3, b3, o3)
            chunk(m4, b4, o4)

    return sk(dst_idx, *msgs, *bases)


def _to_imajor(sph):
    n = sph.shape[0]
    l1 = sph[:, 128:320].reshape(n, 64, 3).transpose(0, 2, 1).reshape(n, 192)
    l2 = sph[:, 320:480].reshape(n, 32, 5).transpose(0, 2, 1).reshape(n, 160)
    return jnp.concatenate([sph[:, :128], l1, l2], axis=1)


def _from_imajor(sph):
    n = sph.shape[0]
    l1 = sph[:, 128:320].reshape(n, 3, 64).transpose(0, 2, 1).reshape(n, 192)
    l2 = sph[:, 320:480].reshape(n, 5, 32).transpose(0, 2, 1).reshape(n, 160)
    return jnp.concatenate([sph[:, :128], l1, l2], axis=1)


def kernel(x_scalar, x_spherical, rbf, rsh, W1, b1, W2, b2, rbf_w, ln_g, ln_b,
           o3_w, o3_b, tp_w, edge_index):
    col_w = jnp.concatenate([
        o3_w[:128],
        jnp.repeat(o3_w[128:192], 3),
        jnp.repeat(o3_w[192:224], 5)]).reshape(1, EDGE_DIM)
    col_b = jnp.concatenate(
        [o3_b, jnp.zeros((EDGE_DIM - 128,), jnp.float32)]).reshape(1, EDGE_DIM)

    xs = jnp.pad(x_scalar, ((0, N_PAD - N_NODES), (0, 0)))
    xsp = jnp.pad(x_spherical, ((0, N_PAD - N_NODES), (0, 0)))
    scalar_in, scalar_out, sph_in = _node_stage(
        xs, xsp, W1, b1, W2, b2, ln_g, ln_b, col_w, col_b)
    sph_in_im = _to_imajor(sph_in)

    W0c, W1c, W2c = _prep_tp_weights(tp_w)

    src = edge_index[1]
    dst = edge_index[0]
    sc_tbl = jnp.pad(scalar_out, ((0, 0), (0, HID_P - HIDDEN)))
    sp_tbl = jnp.pad(sph_in_im, ((0, 0), (0, SPH_P - EDGE_DIM)))
    gsc, gsp = _sc_gather(src, sc_tbl, sp_tbl)

    rsh_t = jnp.pad(rsh.T, ((0, 16 - SPH_DIM), (0, 0)))
    msgs = _edge_stage(gsc, gsp, rbf, rsh_t, rbf_w, W0c.T, W1c.T, W2c.T)

    sp_im_p = jnp.pad(sph_in_im, ((0, 0), (0, SPH_P - EDGE_DIM)))
    acc = _sc_scatter(dst, msgs,
                      (scalar_in, sp_im_p[:, 0:128], sp_im_p[:, 128:256],
                       sp_im_p[:, 256:384], sp_im_p[:, 384:512]))
    new_scalar = acc[0][:N_NODES]
    new_sph_im = jnp.concatenate(acc[1:], axis=1)[:N_NODES, :EDGE_DIM]
    return new_scalar, _from_imajor(new_sph_im)


# speedup vs baseline: 5.6453x; 1.1911x over previous
import functools
from math import factorial

import jax
import jax.numpy as jnp
import numpy as np
from jax.experimental import pallas as pl
from jax.experimental.pallas import tpu as pltpu
from jax.experimental.pallas import tpu_sc as plsc

NODE_DIM = 128
NUM_BASIS = 20
IRREPS = [(128, 0), (64, 1), (32, 2)]
SPH = [(1, 0), (1, 1), (1, 2)]
NUM_IRREPS = sum(m for m, _ in IRREPS)
EDGE_DIM = sum(m * (2 * l + 1) for m, l in IRREPS)
SPH_DIM = sum(m * (2 * l + 1) for m, l in SPH)
HIDDEN = NODE_DIM + NUM_IRREPS
N_NODES = 10000
N_EDGES = 160000
N_PAD = 10240

NODE_BLK = 128
EDGE_BLK = 256


def _su2_cg(j1, j2, j3, m1, m2, m3):
    if m3 != m1 + m2:
        return 0.0
    vmin = int(max(-j1 + j2 + m3, -j1 + m1, 0))
    vmax = int(min(j2 + j3 + m1, j3 - j1 + j2, j3 + m3))

    def f(n):
        return float(factorial(round(n)))

    C = ((2 * j3 + 1) * f(j3 + j1 - j2) * f(j3 - j1 + j2) * f(j1 + j2 - j3) / f(j1 + j2 + j3 + 1)
         * f(j3 + m3) * f(j3 - m3) / (f(j1 - m1) * f(j1 + m1) * f(j2 - m2) * f(j2 + m2))) ** 0.5
    S = 0.0
    for v in range(vmin, vmax + 1):
        S += (-1.0) ** (v + j2 + m2) / f(v) * f(j2 + j3 + m1 - v) * f(j1 - m1 + v) / (
            f(j3 - j1 + j2 - v) * f(j3 + m3 - v) * f(v + j1 - j2 - m3))
    return C * S


def _su2_cg_tensor(l1, l2, l3):
    C = np.zeros((2 * l1 + 1, 2 * l2 + 1, 2 * l3 + 1))
    for m1 in range(-l1, l1 + 1):
        for m2 in range(-l2, l2 + 1):
            m3 = m1 + m2
            if abs(m3) <= l3:
                C[m1 + l1, m2 + l2, m3 + l3] = _su2_cg(l1, l2, l3, m1, m2, m3)
    return C


def _q_mat(l):
    q = np.zeros((2 * l + 1, 2 * l + 1), dtype=complex)
    for m in range(-l, 0):
        q[l + m, l + abs(m)] = 1 / 2 ** 0.5
        q[l + m, l - abs(m)] = -1j / 2 ** 0.5
    q[l, l] = 1.0
    for m in range(1, l + 1):
        q[l + m, l + abs(m)] = (-1) ** m / 2 ** 0.5
        q[l + m, l - abs(m)] = 1j * (-1) ** m / 2 ** 0.5
    return (-1j) ** l * q


def _w3j(l1, l2, l3):
    C = _su2_cg_tensor(l1, l2, l3).astype(complex)
    Q1, Q2, Q3 = _q_mat(l1), _q_mat(l2), _q_mat(l3)
    C = np.einsum('ij,kl,mn,ikm->jln', Q1, Q2, np.conj(Q3), C)
    re, im = np.real(C), np.imag(C)
    C = re if np.abs(re).sum() >= np.abs(im).sum() else im
    n = np.linalg.norm(C)
    return C / n if n > 0 else C


_PATHS = []
for _i1, (_m1, _l1) in enumerate(IRREPS):
    for _i2, (_m2, _l2) in enumerate(SPH):
        for _io, (_mo, _l3) in enumerate(IRREPS):
            if abs(_l1 - _l2) <= _l3 <= _l1 + _l2:
                _PATHS.append((_i1, _i2, _io))
_FAN_IN = [0] * len(IRREPS)
for (_i1, _i2, _io) in _PATHS:
    _FAN_IN[_io] += IRREPS[_i1][0] * SPH[_i2][0]
_W3J = {}
for (_i1, _i2, _io) in _PATHS:
    _k = (IRREPS[_i1][1], SPH[_i2][1], IRREPS[_io][1])
    if _k not in _W3J:
        _W3J[_k] = _w3j(*_k)

_YW = {0: 0, 1: 0, 2: 0}
_COMBO = []
for (_i1, _i2, _io) in _PATHS:
    _m1, _l1 = IRREPS[_i1]
    _, _l2 = SPH[_i2]
    _mo, _l3 = IRREPS[_io]
    _cg = _W3J[(_l1, _l2, _l3)]
    _terms = {}
    for _i in range(2 * _l1 + 1):
        for _j in range(2 * _l2 + 1):
            for _kk in range(2 * _l3 + 1):
                _c = _cg[_i, _j, _kk]
                if abs(_c) > 1e-12:
                    _terms.setdefault((_i, _kk), []).append((_j, float(_c)))
    _COMBO.append((_l1, _l2, _io, _YW[_l1], _mo, _terms))
    _YW[_l1] += _mo

_RBASE = {0: 0, 1: 1, 2: 4}


def _prep_tp_weights(tp_w):
    groups = {0: [], 1: [], 2: []}
    off = 0
    for (i1, i2, io) in _PATHS:
        m1, l1 = IRREPS[i1]
        mo, l3 = IRREPS[io]
        w = tp_w[off:off + m1 * mo].reshape(m1, mo)
        off += m1 * mo
        alpha = (2 * l3 + 1) ** 0.5 / _FAN_IN[io] ** 0.5
        groups[l1].append(w * alpha)
    return (jnp.concatenate(groups[0], axis=1),
            jnp.concatenate(groups[1], axis=1),
            jnp.concatenate(groups[2], axis=1))


def _node_body(xs_ref, xsp_ref, w1_ref, b1_ref, w2_ref, b2_ref, lng_ref,
               lnb_ref, colw_ref, colb_ref, sin_ref, sout_ref, sphn_ref):
    f32 = jnp.float32
    bf = jnp.bfloat16
    x = xs_ref[...]
    mu = jnp.mean(x, axis=1, keepdims=True)
    xc = x - mu
    var = jnp.mean(xc * xc, axis=1, keepdims=True)
    sin = xc / jnp.sqrt(var + 1e-5) * lng_ref[...] + lnb_ref[...]
    sin_ref[...] = sin
    h = jnp.dot(sin.astype(bf), w1_ref[...].astype(bf),
                preferred_element_type=f32) + b1_ref[...]
    h = h * jax.nn.sigmoid(h)
    sout_ref[...] = jnp.dot(h.astype(bf), w2_ref[...].astype(bf),
                            preferred_element_type=f32) + b2_ref[...]

    sp = xsp_ref[...]
    v = sp[:, :128]
    mu0 = jnp.mean(v, axis=1, keepdims=True)
    v = v - mu0
    o0 = v / jnp.sqrt(jnp.mean(v * v, axis=1, keepdims=True) + 1e-5)
    blk1 = sp[:, 128:320]
    n1 = jnp.sum(blk1 * blk1, axis=1, keepdims=True) * (1.0 / 64.0)
    o1 = blk1 / jnp.sqrt(n1 + 1e-5)
    blk2 = sp[:, 320:480]
    n2 = jnp.sum(blk2 * blk2, axis=1, keepdims=True) * (1.0 / 32.0)
    o2 = blk2 / jnp.sqrt(n2 + 1e-5)
    out = jnp.concatenate([o0, o1, o2], axis=1) * colw_ref[...] + colb_ref[...]
    sphn_ref[...] = out


def _node_stage(x_scalar, x_spherical, W1, b1, W2, b2, ln_g, ln_b, col_w, col_b):
    nblk = N_PAD // NODE_BLK
    full = lambda shape: pl.BlockSpec(shape, lambda i: (0, 0))
    row = lambda n: pl.BlockSpec((NODE_BLK, n), lambda i: (i, 0))
    return pl.pallas_call(
        _node_body,
        grid=(nblk,),
        in_specs=[row(NODE_DIM), row(EDGE_DIM),
                  full((NODE_DIM, NODE_DIM)), full((1, NODE_DIM)),
                  full((NODE_DIM, HIDDEN)), full((1, HIDDEN)),
                  full((1, NODE_DIM)), full((1, NODE_DIM)),
                  full((1, EDGE_DIM)), full((1, EDGE_DIM))],
        out_specs=[row(NODE_DIM), row(HIDDEN), row(EDGE_DIM)],
        out_shape=[jax.ShapeDtypeStruct((N_PAD, NODE_DIM), jnp.float32),
                   jax.ShapeDtypeStruct((N_PAD, HIDDEN), jnp.float32),
                   jax.ShapeDtypeStruct((N_PAD, EDGE_DIM), jnp.float32)],
    )(x_scalar, x_spherical, W1, b1.reshape(1, -1), W2, b2.reshape(1, -1),
      ln_g.reshape(1, -1), ln_b.reshape(1, -1), col_w, col_b)


def _edge_body(gsc_ref, gsp_ref, rbf_ref, rsht_ref, rbfw_ref, w0_ref, w1_ref,
               w2_ref, msc_ref, ms0_ref, ms1_ref, ms2_ref, ms3_ref):
    f32 = jnp.float32
    bf = jnp.bfloat16
    fw = jnp.dot(rbf_ref[...].astype(bf), rbfw_ref[...].astype(bf),
                 preferred_element_type=f32)
    fo = gsc_ref[...][:, :HIDDEN] * fw
    msc_ref[...] = fo[:, NUM_IRREPS:]
    gt = fo[:, :NUM_IRREPS].T
    gsp = gsp_ref[...][:, :EDGE_DIM].T

    g0 = gt[:128, :]
    g1 = gt[128:192, :]
    g2 = gt[192:224, :]
    x0 = (gsp[:128, :] * g0).astype(bf)
    x1 = [(gsp[128 + 64 * i:128 + 64 * (i + 1), :] * g1).astype(bf)
          for i in range(3)]
    x2 = [(gsp[320 + 32 * i:320 + 32 * (i + 1), :] * g2).astype(bf)
          for i in range(5)]
    W0 = w0_ref[...].astype(bf)
    W1 = w1_ref[...].astype(bf)
    W2 = w2_ref[...].astype(bf)
    Y = {0: [jnp.dot(W0, x0, preferred_element_type=f32)],
         1: [jnp.dot(W1, x, preferred_element_type=f32) for x in x1],
         2: [jnp.dot(W2, x, preferred_element_type=f32) for x in x2]}

    rsh = rsht_ref[...]
    acc = {0: [None], 1: [None] * 3, 2: [None] * 5}
    for (l1, l2, io, yoff, mo, terms) in _COMBO:
        rbase = _RBASE[l2]
        for (i, k), jl in sorted(terms.items()):
            kv = None
            for (j, c) in jl:
                t = c * rsh[rbase + j:rbase + j + 1, :]
                kv = t if kv is None else kv + t
            contrib = kv * Y[l1][i][yoff:yoff + mo, :]
            acc[io][k] = contrib if acc[io][k] is None else acc[io][k] + contrib
    sph = jnp.concatenate(acc[0] + acc[1] + acc[2], axis=0)
    ms0_ref[...] = sph[0:128, :].T
    ms1_ref[...] = sph[128:256, :].T
    ms2_ref[...] = sph[256:384, :].T
    ms3_ref[...] = jnp.concatenate(
        [sph[384:480, :], jnp.zeros_like(sph[:32, :])], axis=0).T


def _edge_stage(gsc, gsp, rbf, rsh_t, rbf_w, W0t, W1t, W2t):
    n_edges = gsc.shape[0]
    nblk = n_edges // EDGE_BLK
    full = lambda shape: pl.BlockSpec(shape, lambda i: (0, 0))
    row = lambda n: pl.BlockSpec((EDGE_BLK, n), lambda i: (i, 0))
    col = lambda n: pl.BlockSpec((n, EDGE_BLK), lambda i: (0, i))
    return pl.pallas_call(
        _edge_body,
        grid=(nblk,),
        in_specs=[row(HID_P), row(SPH_P), row(NUM_BASIS), col(16),
                  full((NUM_BASIS, HIDDEN)), full((224, 128)),
                  full((384, 64)), full((352, 32))],
        out_specs=[row(128)] * 5,
        out_shape=[jax.ShapeDtypeStruct((n_edges, 128), jnp.float32)] * 5,
    )(gsc, gsp, rbf, rsh_t, rbf_w, W0t, W1t, W2t)


_SC_MESH = plsc.VectorSubcoreMesh(core_axis_name="c", subcore_axis_name="s",
                                  num_cores=2, num_subcores=16)
_NW = 32
_GC = 128
_NCHUNK = N_EDGES // _GC
HID_P = 384
SPH_P = 512


def _sc_gather(src_idx, scalar_tbl, sph_tbl):
    n_edges = src_idx.shape[0]
    nchunk = n_edges // _GC
    base_chunks = nchunk // _NW
    rem = nchunk % _NW

    @functools.partial(
        pl.kernel,
        out_type=[jax.ShapeDtypeStruct((n_edges, HID_P), jnp.float32),
                  jax.ShapeDtypeStruct((n_edges, SPH_P), jnp.float32)],
        mesh=_SC_MESH,
    )
    def gk(idx_hbm, t1_hbm, t2_hbm, o1_hbm, o2_hbm):
        wid = jax.lax.axis_index("s") * 2 + jax.lax.axis_index("c")
        nmine = base_chunks + jnp.where(wid < rem, 1, 0)

        def phase(tbl, out, width):
            def body(idx_v, rows_v, sem):
                def step(i, _):
                    e0 = (wid + i * _NW) * _GC
                    pltpu.sync_copy(idx_hbm.at[pl.ds(e0, _GC)], idx_v)
                    pltpu.async_copy(tbl.at[idx_v], rows_v, sem).wait()
                    pltpu.sync_copy(rows_v, out.at[pl.ds(e0, _GC)])
                    return 0
                jax.lax.fori_loop(0, nmine, step, 0)
            pl.run_scoped(body,
                          pltpu.VMEM((_GC,), jnp.int32),
                          pltpu.VMEM((_GC, width), jnp.float32),
                          pltpu.SemaphoreType.DMA)

        phase(t1_hbm, o1_hbm, HID_P)
        phase(t2_hbm, o2_hbm, SPH_P)

    return gk(src_idx, scalar_tbl, sph_tbl)


_SCHUNK = 128
_ROWS_PER_TILE = N_PAD // 16


def _sc_scatter(dst_idx, msgs, bases):
    nchunk = dst_idx.shape[0] // _SCHUNK
    base_chunks = nchunk // 16
    rem = nchunk % 16

    @functools.partial(
        pl.kernel,
        out_type=[jax.ShapeDtypeStruct((N_PAD, 128), jnp.float32)] * 5,
        mesh=_SC_MESH,
        scratch_types=[pltpu.VMEM_SHARED((N_PAD, 128), jnp.float32)],
    )
    def sk(dst_hbm, m0, m1, m2, m3, m4, b0, b1, b2, b3, b4,
           o0, o1, o2, o3, o4, acc_s):
        cid = jax.lax.axis_index("c")
        sid = jax.lax.axis_index("s")
        r0 = sid * _ROWS_PER_TILE
        nmine = base_chunks + jnp.where(sid < rem, 1, 0)

        def chunk(m, b, o):
            pltpu.sync_copy(b.at[pl.ds(r0, _ROWS_PER_TILE)],
                            acc_s.at[pl.ds(r0, _ROWS_PER_TILE)])
            plsc.subcore_barrier()

            def body(idx_v, m_v, sem):
                def step(i, _):
                    e0 = (sid + i * 16) * _SCHUNK
                    pltpu.sync_copy(dst_hbm.at[pl.ds(e0, _SCHUNK)], idx_v)
                    pltpu.async_copy(m.at[pl.ds(e0, _SCHUNK)], m_v, sem).wait()
                    pltpu.sync_copy(m_v, acc_s.at[idx_v], add=True)
                    return 0
                jax.lax.fori_loop(0, nmine, step, 0)
            pl.run_scoped(body,
                          pltpu.VMEM((_SCHUNK,), jnp.int32),
                          pltpu.VMEM((_SCHUNK, 128), jnp.float32),
                          pltpu.SemaphoreType.DMA)
            plsc.subcore_barrier()
            pltpu.sync_copy(acc_s.at[pl.ds(r0, _ROWS_PER_TILE)],
                            o.at[pl.ds(r0, _ROWS_PER_TILE)])
            plsc.subcore_barrier()

        @pl.when(cid == 0)
        def _():
            chunk(m0, b0, o0)
            chunk(m1, b1, o1)
            chunk(m2, b2, o2)

        @pl.when(cid == 1)
        def _():
            chunk(m3, b3, o3)
            chunk(m4, b4, o4)

    return sk(dst_idx, *msgs, *bases)


def _to_imajor(sph):
    n = sph.shape[0]
    l1 = sph[:, 128:320].reshape(n, 64, 3).transpose(0, 2, 1).reshape(n, 192)
    l2 = sph[:, 320:480].reshape(n, 32, 5).transpose(0, 2, 1).reshape(n, 160)
    return jnp.concatenate([sph[:, :128], l1, l2], axis=1)


def _from_imajor(sph):
    n = sph.shape[0]
    l1 = sph[:, 128:320].reshape(n, 3, 64).transpose(0, 2, 1).reshape(n, 192)
    l2 = sph[:, 320:480].reshape(n, 5, 32).transpose(0, 2, 1).reshape(n, 160)
    return jnp.concatenate([sph[:, :128], l1, l2], axis=1)


def kernel(x_scalar, x_spherical, rbf, rsh, W1, b1, W2, b2, rbf_w, ln_g, ln_b,
           o3_w, o3_b, tp_w, edge_index):
    col_w = jnp.concatenate([
        o3_w[:128],
        jnp.repeat(o3_w[128:192], 3),
        jnp.repeat(o3_w[192:224], 5)]).reshape(1, EDGE_DIM)
    col_b = jnp.concatenate(
        [o3_b, jnp.zeros((EDGE_DIM - 128,), jnp.float32)]).reshape(1, EDGE_DIM)

    xs = jnp.pad(x_scalar, ((0, N_PAD - N_NODES), (0, 0)))
    xsp = jnp.pad(x_spherical, ((0, N_PAD - N_NODES), (0, 0)))
    scalar_in, scalar_out, sph_in = _node_stage(
        xs, xsp, W1, b1, W2, b2, ln_g, ln_b, col_w, col_b)
    sph_in_im = _to_imajor(sph_in)

    W0c, W1c, W2c = _prep_tp_weights(tp_w)

    src = edge_index[1]
    dst = edge_index[0]
    sc_tbl = jnp.pad(scalar_out, ((0, 0), (0, HID_P - HIDDEN)))
    sp_tbl = jnp.pad(sph_in_im, ((0, 0), (0, SPH_P - EDGE_DIM)))
    rsh_t = jnp.pad(rsh.T, ((0, 16 - SPH_DIM), (0, 0)))

    cuts = [0, 80128, N_EDGES]
    parts = list(zip(cuts[:-1], cuts[1:]))
    gathered = [_sc_gather(src[lo:hi], sc_tbl, sp_tbl) for (lo, hi) in parts]
    tables = (scalar_in, sp_tbl[:, 0:128], sp_tbl[:, 128:256],
              sp_tbl[:, 256:384], sp_tbl[:, 384:512])
    for (lo, hi), (gsc, gsp) in zip(parts, gathered):
        msgs = _edge_stage(gsc, gsp, rbf[lo:hi], rsh_t[:, lo:hi], rbf_w,
                           W0c.T, W1c.T, W2c.T)
        tables = _sc_scatter(dst[lo:hi], msgs, tables)
    new_scalar = tables[0][:N_NODES]
    new_sph_im = jnp.concatenate(tables[1:], axis=1)[:N_NODES, :EDGE_DIM]
    return new_scalar, _from_imajor(new_sph_im)


# speedup vs baseline: 5.8904x; 1.0434x over previous
import functools
from math import factorial

import jax
import jax.numpy as jnp
import numpy as np
from jax.experimental import pallas as pl
from jax.experimental.pallas import tpu as pltpu
from jax.experimental.pallas import tpu_sc as plsc

NODE_DIM = 128
NUM_BASIS = 20
IRREPS = [(128, 0), (64, 1), (32, 2)]
SPH = [(1, 0), (1, 1), (1, 2)]
NUM_IRREPS = sum(m for m, _ in IRREPS)
EDGE_DIM = sum(m * (2 * l + 1) for m, l in IRREPS)
SPH_DIM = sum(m * (2 * l + 1) for m, l in SPH)
HIDDEN = NODE_DIM + NUM_IRREPS
N_NODES = 10000
N_EDGES = 160000
N_PAD = 10240

NODE_BLK = 128
EDGE_BLK = 256


def _su2_cg(j1, j2, j3, m1, m2, m3):
    if m3 != m1 + m2:
        return 0.0
    vmin = int(max(-j1 + j2 + m3, -j1 + m1, 0))
    vmax = int(min(j2 + j3 + m1, j3 - j1 + j2, j3 + m3))

    def f(n):
        return float(factorial(round(n)))

    C = ((2 * j3 + 1) * f(j3 + j1 - j2) * f(j3 - j1 + j2) * f(j1 + j2 - j3) / f(j1 + j2 + j3 + 1)
         * f(j3 + m3) * f(j3 - m3) / (f(j1 - m1) * f(j1 + m1) * f(j2 - m2) * f(j2 + m2))) ** 0.5
    S = 0.0
    for v in range(vmin, vmax + 1):
        S += (-1.0) ** (v + j2 + m2) / f(v) * f(j2 + j3 + m1 - v) * f(j1 - m1 + v) / (
            f(j3 - j1 + j2 - v) * f(j3 + m3 - v) * f(v + j1 - j2 - m3))
    return C * S


def _su2_cg_tensor(l1, l2, l3):
    C = np.zeros((2 * l1 + 1, 2 * l2 + 1, 2 * l3 + 1))
    for m1 in range(-l1, l1 + 1):
        for m2 in range(-l2, l2 + 1):
            m3 = m1 + m2
            if abs(m3) <= l3:
                C[m1 + l1, m2 + l2, m3 + l3] = _su2_cg(l1, l2, l3, m1, m2, m3)
    return C


def _q_mat(l):
    q = np.zeros((2 * l + 1, 2 * l + 1), dtype=complex)
    for m in range(-l, 0):
        q[l + m, l + abs(m)] = 1 / 2 ** 0.5
        q[l + m, l - abs(m)] = -1j / 2 ** 0.5
    q[l, l] = 1.0
    for m in range(1, l + 1):
        q[l + m, l + abs(m)] = (-1) ** m / 2 ** 0.5
        q[l + m, l - abs(m)] = 1j * (-1) ** m / 2 ** 0.5
    return (-1j) ** l * q


def _w3j(l1, l2, l3):
    C = _su2_cg_tensor(l1, l2, l3).astype(complex)
    Q1, Q2, Q3 = _q_mat(l1), _q_mat(l2), _q_mat(l3)
    C = np.einsum('ij,kl,mn,ikm->jln', Q1, Q2, np.conj(Q3), C)
    re, im = np.real(C), np.imag(C)
    C = re if np.abs(re).sum() >= np.abs(im).sum() else im
    n = np.linalg.norm(C)
    return C / n if n > 0 else C


_PATHS = []
for _i1, (_m1, _l1) in enumerate(IRREPS):
    for _i2, (_m2, _l2) in enumerate(SPH):
        for _io, (_mo, _l3) in enumerate(IRREPS):
            if abs(_l1 - _l2) <= _l3 <= _l1 + _l2:
                _PATHS.append((_i1, _i2, _io))
_FAN_IN = [0] * len(IRREPS)
for (_i1, _i2, _io) in _PATHS:
    _FAN_IN[_io] += IRREPS[_i1][0] * SPH[_i2][0]
_W3J = {}
for (_i1, _i2, _io) in _PATHS:
    _k = (IRREPS[_i1][1], SPH[_i2][1], IRREPS[_io][1])
    if _k not in _W3J:
        _W3J[_k] = _w3j(*_k)

_YW = {0: 0, 1: 0, 2: 0}
_COMBO = []
for (_i1, _i2, _io) in _PATHS:
    _m1, _l1 = IRREPS[_i1]
    _, _l2 = SPH[_i2]
    _mo, _l3 = IRREPS[_io]
    _cg = _W3J[(_l1, _l2, _l3)]
    _terms = {}
    for _i in range(2 * _l1 + 1):
        for _j in range(2 * _l2 + 1):
            for _kk in range(2 * _l3 + 1):
                _c = _cg[_i, _j, _kk]
                if abs(_c) > 1e-12:
                    _terms.setdefault((_i, _kk), []).append((_j, float(_c)))
    _COMBO.append((_l1, _l2, _io, _YW[_l1], _mo, _terms))
    _YW[_l1] += _mo

_RBASE = {0: 0, 1: 1, 2: 4}


def _prep_tp_weights(tp_w):
    groups = {0: [], 1: [], 2: []}
    off = 0
    for (i1, i2, io) in _PATHS:
        m1, l1 = IRREPS[i1]
        mo, l3 = IRREPS[io]
        w = tp_w[off:off + m1 * mo].reshape(m1, mo)
        off += m1 * mo
        alpha = (2 * l3 + 1) ** 0.5 / _FAN_IN[io] ** 0.5
        groups[l1].append(w * alpha)
    return (jnp.concatenate(groups[0], axis=1),
            jnp.concatenate(groups[1], axis=1),
            jnp.concatenate(groups[2], axis=1))


def _node_body(xs_ref, xsp_ref, w1_ref, b1_ref, w2_ref, b2_ref, lng_ref,
               lnb_ref, colw_ref, colb_ref, sin_ref, sout_ref, sphn_ref):
    f32 = jnp.float32
    bf = jnp.bfloat16
    x = xs_ref[...]
    mu = jnp.mean(x, axis=1, keepdims=True)
    xc = x - mu
    var = jnp.mean(xc * xc, axis=1, keepdims=True)
    sin = xc / jnp.sqrt(var + 1e-5) * lng_ref[...] + lnb_ref[...]
    sin_ref[...] = sin
    h = jnp.dot(sin.astype(bf), w1_ref[...].astype(bf),
                preferred_element_type=f32) + b1_ref[...]
    h = h * jax.nn.sigmoid(h)
    sout_ref[...] = jnp.dot(h.astype(bf), w2_ref[...].astype(bf),
                            preferred_element_type=f32) + b2_ref[...]

    sp = xsp_ref[...]
    v = sp[:, :128]
    mu0 = jnp.mean(v, axis=1, keepdims=True)
    v = v - mu0
    o0 = v / jnp.sqrt(jnp.mean(v * v, axis=1, keepdims=True) + 1e-5)
    blk1 = sp[:, 128:320]
    n1 = jnp.sum(blk1 * blk1, axis=1, keepdims=True) * (1.0 / 64.0)
    o1 = blk1 / jnp.sqrt(n1 + 1e-5)
    blk2 = sp[:, 320:480]
    n2 = jnp.sum(blk2 * blk2, axis=1, keepdims=True) * (1.0 / 32.0)
    o2 = blk2 / jnp.sqrt(n2 + 1e-5)
    out = jnp.concatenate([o0, o1, o2], axis=1) * colw_ref[...] + colb_ref[...]
    sphn_ref[...] = out


def _node_stage(x_scalar, x_spherical, W1, b1, W2, b2, ln_g, ln_b, col_w, col_b):
    nblk = N_PAD // NODE_BLK
    full = lambda shape: pl.BlockSpec(shape, lambda i: (0, 0))
    row = lambda n: pl.BlockSpec((NODE_BLK, n), lambda i: (i, 0))
    return pl.pallas_call(
        _node_body,
        grid=(nblk,),
        in_specs=[row(NODE_DIM), row(EDGE_DIM),
                  full((NODE_DIM, NODE_DIM)), full((1, NODE_DIM)),
                  full((NODE_DIM, HIDDEN)), full((1, HIDDEN)),
                  full((1, NODE_DIM)), full((1, NODE_DIM)),
                  full((1, EDGE_DIM)), full((1, EDGE_DIM))],
        out_specs=[row(NODE_DIM), row(HIDDEN), row(EDGE_DIM)],
        out_shape=[jax.ShapeDtypeStruct((N_PAD, NODE_DIM), jnp.float32),
                   jax.ShapeDtypeStruct((N_PAD, HIDDEN), jnp.float32),
                   jax.ShapeDtypeStruct((N_PAD, EDGE_DIM), jnp.float32)],
    )(x_scalar, x_spherical, W1, b1.reshape(1, -1), W2, b2.reshape(1, -1),
      ln_g.reshape(1, -1), ln_b.reshape(1, -1), col_w, col_b)


def _edge_body(gsc_ref, gsp_ref, rbf_ref, rsht_ref, rbfw_ref, w0_ref, w1_ref,
               w2_ref, msc_ref, ms0_ref, ms1_ref, ms2_ref, ms3_ref):
    f32 = jnp.float32
    bf = jnp.bfloat16
    fw = jnp.dot(rbf_ref[...].astype(bf), rbfw_ref[...].astype(bf),
                 preferred_element_type=f32)
    fo = gsc_ref[...][:, :HIDDEN] * fw
    msc_ref[...] = fo[:, NUM_IRREPS:]
    gt = fo[:, :NUM_IRREPS].T
    gsp = gsp_ref[...][:, :EDGE_DIM].T

    g0 = gt[:128, :]
    g1 = gt[128:192, :]
    g2 = gt[192:224, :]
    x0 = (gsp[:128, :] * g0).astype(bf)
    x1 = [(gsp[128 + 64 * i:128 + 64 * (i + 1), :] * g1).astype(bf)
          for i in range(3)]
    x2 = [(gsp[320 + 32 * i:320 + 32 * (i + 1), :] * g2).astype(bf)
          for i in range(5)]
    W0 = w0_ref[...].astype(bf)
    W1 = w1_ref[...].astype(bf)
    W2 = w2_ref[...].astype(bf)
    Y = {0: [jnp.dot(W0, x0, preferred_element_type=f32)],
         1: [jnp.dot(W1, x, preferred_element_type=f32) for x in x1],
         2: [jnp.dot(W2, x, preferred_element_type=f32) for x in x2]}

    rsh = rsht_ref[...]
    acc = {0: [None], 1: [None] * 3, 2: [None] * 5}
    for (l1, l2, io, yoff, mo, terms) in _COMBO:
        rbase = _RBASE[l2]
        for (i, k), jl in sorted(terms.items()):
            kv = None
            for (j, c) in jl:
                t = c * rsh[rbase + j:rbase + j + 1, :]
                kv = t if kv is None else kv + t
            contrib = kv * Y[l1][i][yoff:yoff + mo, :]
            acc[io][k] = contrib if acc[io][k] is None else acc[io][k] + contrib
    sph = jnp.concatenate(acc[0] + acc[1] + acc[2], axis=0)
    ms0_ref[...] = sph[0:128, :].T
    ms1_ref[...] = sph[128:256, :].T
    ms2_ref[...] = sph[256:384, :].T
    ms3_ref[...] = jnp.concatenate(
        [sph[384:480, :], jnp.zeros_like(sph[:32, :])], axis=0).T


def _edge_stage(gsc, gsp, rbf, rsh_t, rbf_w, W0t, W1t, W2t):
    n_edges = gsc.shape[0]
    nblk = n_edges // EDGE_BLK
    full = lambda shape: pl.BlockSpec(shape, lambda i: (0, 0))
    row = lambda n: pl.BlockSpec((EDGE_BLK, n), lambda i: (i, 0))
    col = lambda n: pl.BlockSpec((n, EDGE_BLK), lambda i: (0, i))
    return pl.pallas_call(
        _edge_body,
        grid=(nblk,),
        in_specs=[row(HID_P), row(SPH_P), row(NUM_BASIS), col(16),
                  full((NUM_BASIS, HIDDEN)), full((224, 128)),
                  full((384, 64)), full((352, 32))],
        out_specs=[row(128)] * 5,
        out_shape=[jax.ShapeDtypeStruct((n_edges, 128), jnp.float32)] * 5,
    )(gsc, gsp, rbf, rsh_t, rbf_w, W0t, W1t, W2t)


_SC_MESH = plsc.VectorSubcoreMesh(core_axis_name="c", subcore_axis_name="s",
                                  num_cores=2, num_subcores=16)
_NW = 32
_GC = 128
_NCHUNK = N_EDGES // _GC
HID_P = 384
SPH_P = 512


def _sc_gather(src_idx, scalar_tbl, sph_tbl):
    n_edges = src_idx.shape[0]
    nchunk = n_edges // _GC
    base_chunks = nchunk // _NW
    rem = nchunk % _NW

    @functools.partial(
        pl.kernel,
        out_type=[jax.ShapeDtypeStruct((n_edges, HID_P), jnp.float32),
                  jax.ShapeDtypeStruct((n_edges, SPH_P), jnp.float32)],
        mesh=_SC_MESH,
    )
    def gk(idx_hbm, t1_hbm, t2_hbm, o1_hbm, o2_hbm):
        wid = jax.lax.axis_index("s") * 2 + jax.lax.axis_index("c")
        nmine = base_chunks + jnp.where(wid < rem, 1, 0)

        def phase(tbl, out, width):
            def body(idx_v, rows_v, sem):
                def step(i, _):
                    e0 = (wid + i * _NW) * _GC
                    pltpu.sync_copy(idx_hbm.at[pl.ds(e0, _GC)], idx_v)
                    pltpu.async_copy(tbl.at[idx_v], rows_v, sem).wait()
                    pltpu.sync_copy(rows_v, out.at[pl.ds(e0, _GC)])
                    return 0
                jax.lax.fori_loop(0, nmine, step, 0)
            pl.run_scoped(body,
                          pltpu.VMEM((_GC,), jnp.int32),
                          pltpu.VMEM((_GC, width), jnp.float32),
                          pltpu.SemaphoreType.DMA)

        phase(t1_hbm, o1_hbm, HID_P)
        phase(t2_hbm, o2_hbm, SPH_P)

    return gk(src_idx, scalar_tbl, sph_tbl)


_SCHUNK = 128
_ROWS_PER_TILE = N_PAD // 16


def _sc_scatter(dst_idx, msgs, bases):
    nsuper = dst_idx.shape[0] // (2 * _SCHUNK)
    base_chunks = nsuper // 16
    rem = nsuper % 16
    dst2 = dst_idx.reshape(-1, _SCHUNK)

    @functools.partial(
        pl.kernel,
        out_type=[jax.ShapeDtypeStruct((N_PAD, 128), jnp.float32)] * 5,
        mesh=_SC_MESH,
        scratch_types=[pltpu.VMEM_SHARED((N_PAD, 128), jnp.float32)],
    )
    def sk(dst_hbm, m0, m1, m2, m3, m4, b0, b1, b2, b3, b4,
           o0, o1, o2, o3, o4, acc_s):
        cid = jax.lax.axis_index("c")
        sid = jax.lax.axis_index("s")
        r0 = sid * _ROWS_PER_TILE
        nmine = base_chunks + jnp.where(sid < rem, 1, 0)

        def chunk(m, b, o):
            pltpu.sync_copy(b.at[pl.ds(r0, _ROWS_PER_TILE)],
                            acc_s.at[pl.ds(r0, _ROWS_PER_TILE)])
            plsc.subcore_barrier()

            def body(idx_v, m_v, lsem, asem):
                def step(j, _):
                    sci = sid + j * 16
                    ca = pltpu.async_copy(dst_hbm.at[pl.ds(sci * 2, 2)],
                                          idx_v, lsem)
                    cb = pltpu.async_copy(m.at[pl.ds(sci * 2 * _SCHUNK,
                                                     2 * _SCHUNK)], m_v, lsem)
                    ca.wait()
                    cb.wait()
                    a0 = pltpu.async_copy(m_v.at[pl.ds(0, _SCHUNK)],
                                          acc_s.at[idx_v.at[0]], asem,
                                          add=True)
                    a1 = pltpu.async_copy(m_v.at[pl.ds(_SCHUNK, _SCHUNK)],
                                          acc_s.at[idx_v.at[1]], asem,
                                          add=True)
                    a0.wait()
                    a1.wait()
                    return 0
                jax.lax.fori_loop(0, nmine, step, 0)
            pl.run_scoped(body,
                          pltpu.VMEM((2, _SCHUNK), jnp.int32),
                          pltpu.VMEM((2 * _SCHUNK, 128), jnp.float32),
                          pltpu.SemaphoreType.DMA,
                          pltpu.SemaphoreType.DMA)
            plsc.subcore_barrier()
            pltpu.sync_copy(acc_s.at[pl.ds(r0, _ROWS_PER_TILE)],
                            o.at[pl.ds(r0, _ROWS_PER_TILE)])
            plsc.subcore_barrier()

        @pl.when(cid == 0)
        def _():
            chunk(m0, b0, o0)
            chunk(m1, b1, o1)
            chunk(m2, b2, o2)

        @pl.when(cid == 1)
        def _():
            chunk(m3, b3, o3)
            chunk(m4, b4, o4)

    return sk(dst2, *msgs, *bases)


def _to_imajor(sph):
    n = sph.shape[0]
    l1 = sph[:, 128:320].reshape(n, 64, 3).transpose(0, 2, 1).reshape(n, 192)
    l2 = sph[:, 320:480].reshape(n, 32, 5).transpose(0, 2, 1).reshape(n, 160)
    return jnp.concatenate([sph[:, :128], l1, l2], axis=1)


def _from_imajor(sph):
    n = sph.shape[0]
    l1 = sph[:, 128:320].reshape(n, 3, 64).transpose(0, 2, 1).reshape(n, 192)
    l2 = sph[:, 320:480].reshape(n, 5, 32).transpose(0, 2, 1).reshape(n, 160)
    return jnp.concatenate([sph[:, :128], l1, l2], axis=1)


def kernel(x_scalar, x_spherical, rbf, rsh, W1, b1, W2, b2, rbf_w, ln_g, ln_b,
           o3_w, o3_b, tp_w, edge_index):
    col_w = jnp.concatenate([
        o3_w[:128],
        jnp.repeat(o3_w[128:192], 3),
        jnp.repeat(o3_w[192:224], 5)]).reshape(1, EDGE_DIM)
    col_b = jnp.concatenate(
        [o3_b, jnp.zeros((EDGE_DIM - 128,), jnp.float32)]).reshape(1, EDGE_DIM)

    xs = jnp.pad(x_scalar, ((0, N_PAD - N_NODES), (0, 0)))
    xsp = jnp.pad(x_spherical, ((0, N_PAD - N_NODES), (0, 0)))
    scalar_in, scalar_out, sph_in = _node_stage(
        xs, xsp, W1, b1, W2, b2, ln_g, ln_b, col_w, col_b)
    sph_in_im = _to_imajor(sph_in)

    W0c, W1c, W2c = _prep_tp_weights(tp_w)

    src = edge_index[1]
    dst = edge_index[0]
    sc_tbl = jnp.pad(scalar_out, ((0, 0), (0, HID_P - HIDDEN)))
    sp_tbl = jnp.pad(sph_in_im, ((0, 0), (0, SPH_P - EDGE_DIM)))
    rsh_t = jnp.pad(rsh.T, ((0, 16 - SPH_DIM), (0, 0)))

    cuts = [0, 80128, N_EDGES]
    parts = list(zip(cuts[:-1], cuts[1:]))
    gathered = [_sc_gather(src[lo:hi], sc_tbl, sp_tbl) for (lo, hi) in parts]
    tables = (scalar_in, sp_tbl[:, 0:128], sp_tbl[:, 128:256],
              sp_tbl[:, 256:384], sp_tbl[:, 384:512])
    for (lo, hi), (gsc, gsp) in zip(parts, gathered):
        msgs = _edge_stage(gsc, gsp, rbf[lo:hi], rsh_t[:, lo:hi], rbf_w,
                           W0c.T, W1c.T, W2c.T)
        tables = _sc_scatter(dst[lo:hi], msgs, tables)
    new_scalar = tables[0][:N_NODES]
    new_sph_im = jnp.concatenate(tables[1:], axis=1)[:N_NODES, :EDGE_DIM]
    return new_scalar, _from_imajor(new_sph_im)


# speedup vs baseline: 6.3486x; 1.0778x over previous
import functools
from math import factorial

import jax
import jax.numpy as jnp
import numpy as np
from jax.experimental import pallas as pl
from jax.experimental.pallas import tpu as pltpu
from jax.experimental.pallas import tpu_sc as plsc

NODE_DIM = 128
NUM_BASIS = 20
IRREPS = [(128, 0), (64, 1), (32, 2)]
SPH = [(1, 0), (1, 1), (1, 2)]
NUM_IRREPS = sum(m for m, _ in IRREPS)
EDGE_DIM = sum(m * (2 * l + 1) for m, l in IRREPS)
SPH_DIM = sum(m * (2 * l + 1) for m, l in SPH)
HIDDEN = NODE_DIM + NUM_IRREPS
N_NODES = 10000
N_EDGES = 160000
N_PAD = 10240

NODE_BLK = 128
EDGE_BLK = 256


def _su2_cg(j1, j2, j3, m1, m2, m3):
    if m3 != m1 + m2:
        return 0.0
    vmin = int(max(-j1 + j2 + m3, -j1 + m1, 0))
    vmax = int(min(j2 + j3 + m1, j3 - j1 + j2, j3 + m3))

    def f(n):
        return float(factorial(round(n)))

    C = ((2 * j3 + 1) * f(j3 + j1 - j2) * f(j3 - j1 + j2) * f(j1 + j2 - j3) / f(j1 + j2 + j3 + 1)
         * f(j3 + m3) * f(j3 - m3) / (f(j1 - m1) * f(j1 + m1) * f(j2 - m2) * f(j2 + m2))) ** 0.5
    S = 0.0
    for v in range(vmin, vmax + 1):
        S += (-1.0) ** (v + j2 + m2) / f(v) * f(j2 + j3 + m1 - v) * f(j1 - m1 + v) / (
            f(j3 - j1 + j2 - v) * f(j3 + m3 - v) * f(v + j1 - j2 - m3))
    return C * S


def _su2_cg_tensor(l1, l2, l3):
    C = np.zeros((2 * l1 + 1, 2 * l2 + 1, 2 * l3 + 1))
    for m1 in range(-l1, l1 + 1):
        for m2 in range(-l2, l2 + 1):
            m3 = m1 + m2
            if abs(m3) <= l3:
                C[m1 + l1, m2 + l2, m3 + l3] = _su2_cg(l1, l2, l3, m1, m2, m3)
    return C


def _q_mat(l):
    q = np.zeros((2 * l + 1, 2 * l + 1), dtype=complex)
    for m in range(-l, 0):
        q[l + m, l + abs(m)] = 1 / 2 ** 0.5
        q[l + m, l - abs(m)] = -1j / 2 ** 0.5
    q[l, l] = 1.0
    for m in range(1, l + 1):
        q[l + m, l + abs(m)] = (-1) ** m / 2 ** 0.5
        q[l + m, l - abs(m)] = 1j * (-1) ** m / 2 ** 0.5
    return (-1j) ** l * q


def _w3j(l1, l2, l3):
    C = _su2_cg_tensor(l1, l2, l3).astype(complex)
    Q1, Q2, Q3 = _q_mat(l1), _q_mat(l2), _q_mat(l3)
    C = np.einsum('ij,kl,mn,ikm->jln', Q1, Q2, np.conj(Q3), C)
    re, im = np.real(C), np.imag(C)
    C = re if np.abs(re).sum() >= np.abs(im).sum() else im
    n = np.linalg.norm(C)
    return C / n if n > 0 else C


_PATHS = []
for _i1, (_m1, _l1) in enumerate(IRREPS):
    for _i2, (_m2, _l2) in enumerate(SPH):
        for _io, (_mo, _l3) in enumerate(IRREPS):
            if abs(_l1 - _l2) <= _l3 <= _l1 + _l2:
                _PATHS.append((_i1, _i2, _io))
_FAN_IN = [0] * len(IRREPS)
for (_i1, _i2, _io) in _PATHS:
    _FAN_IN[_io] += IRREPS[_i1][0] * SPH[_i2][0]
_W3J = {}
for (_i1, _i2, _io) in _PATHS:
    _k = (IRREPS[_i1][1], SPH[_i2][1], IRREPS[_io][1])
    if _k not in _W3J:
        _W3J[_k] = _w3j(*_k)

_YW = {0: 0, 1: 0, 2: 0}
_COMBO = []
for (_i1, _i2, _io) in _PATHS:
    _m1, _l1 = IRREPS[_i1]
    _, _l2 = SPH[_i2]
    _mo, _l3 = IRREPS[_io]
    _cg = _W3J[(_l1, _l2, _l3)]
    _terms = {}
    for _i in range(2 * _l1 + 1):
        for _j in range(2 * _l2 + 1):
            for _kk in range(2 * _l3 + 1):
                _c = _cg[_i, _j, _kk]
                if abs(_c) > 1e-12:
                    _terms.setdefault((_i, _kk), []).append((_j, float(_c)))
    _COMBO.append((_l1, _l2, _io, _YW[_l1], _mo, _terms))
    _YW[_l1] += _mo

_RBASE = {0: 0, 1: 1, 2: 4}


def _prep_tp_weights(tp_w):
    groups = {0: [], 1: [], 2: []}
    off = 0
    for (i1, i2, io) in _PATHS:
        m1, l1 = IRREPS[i1]
        mo, l3 = IRREPS[io]
        w = tp_w[off:off + m1 * mo].reshape(m1, mo)
        off += m1 * mo
        alpha = (2 * l3 + 1) ** 0.5 / _FAN_IN[io] ** 0.5
        groups[l1].append(w * alpha)
    return (jnp.concatenate(groups[0], axis=1),
            jnp.concatenate(groups[1], axis=1),
            jnp.concatenate(groups[2], axis=1))


def _node_body(xs_ref, xsp_ref, w1_ref, b1_ref, w2_ref, b2_ref, lng_ref,
               lnb_ref, colw_ref, colb_ref, sin_ref, sout_ref, sphn_ref):
    f32 = jnp.float32
    bf = jnp.bfloat16
    x = xs_ref[...]
    mu = jnp.mean(x, axis=1, keepdims=True)
    xc = x - mu
    var = jnp.mean(xc * xc, axis=1, keepdims=True)
    sin = xc / jnp.sqrt(var + 1e-5) * lng_ref[...] + lnb_ref[...]
    sin_ref[...] = sin
    h = jnp.dot(sin.astype(bf), w1_ref[...].astype(bf),
                preferred_element_type=f32) + b1_ref[...]
    h = h * jax.nn.sigmoid(h)
    sout_ref[...] = jnp.dot(h.astype(bf), w2_ref[...].astype(bf),
                            preferred_element_type=f32) + b2_ref[...]

    sp = xsp_ref[...]
    v = sp[:, :128]
    mu0 = jnp.mean(v, axis=1, keepdims=True)
    v = v - mu0
    o0 = v / jnp.sqrt(jnp.mean(v * v, axis=1, keepdims=True) + 1e-5)
    blk1 = sp[:, 128:320]
    n1 = jnp.sum(blk1 * blk1, axis=1, keepdims=True) * (1.0 / 64.0)
    o1 = blk1 / jnp.sqrt(n1 + 1e-5)
    blk2 = sp[:, 320:480]
    n2 = jnp.sum(blk2 * blk2, axis=1, keepdims=True) * (1.0 / 32.0)
    o2 = blk2 / jnp.sqrt(n2 + 1e-5)
    out = jnp.concatenate([o0, o1, o2], axis=1) * colw_ref[...] + colb_ref[...]
    sphn_ref[...] = out


def _node_stage(x_scalar, x_spherical, W1, b1, W2, b2, ln_g, ln_b, col_w, col_b):
    nblk = N_PAD // NODE_BLK
    full = lambda shape: pl.BlockSpec(shape, lambda i: (0, 0))
    row = lambda n: pl.BlockSpec((NODE_BLK, n), lambda i: (i, 0))
    return pl.pallas_call(
        _node_body,
        grid=(nblk,),
        in_specs=[row(NODE_DIM), row(EDGE_DIM),
                  full((NODE_DIM, NODE_DIM)), full((1, NODE_DIM)),
                  full((NODE_DIM, HIDDEN)), full((1, HIDDEN)),
                  full((1, NODE_DIM)), full((1, NODE_DIM)),
                  full((1, EDGE_DIM)), full((1, EDGE_DIM))],
        out_specs=[row(NODE_DIM), row(HIDDEN), row(EDGE_DIM)],
        out_shape=[jax.ShapeDtypeStruct((N_PAD, NODE_DIM), jnp.float32),
                   jax.ShapeDtypeStruct((N_PAD, HIDDEN), jnp.float32),
                   jax.ShapeDtypeStruct((N_PAD, EDGE_DIM), jnp.float32)],
    )(x_scalar, x_spherical, W1, b1.reshape(1, -1), W2, b2.reshape(1, -1),
      ln_g.reshape(1, -1), ln_b.reshape(1, -1), col_w, col_b)


def _edge_body(gsc_ref, gsp_ref, rbf_ref, rsht_ref, rbfw_ref, w0_ref, w1_ref,
               w2_ref, msc_ref, ms0_ref, ms1_ref, ms2_ref, ms3_ref):
    f32 = jnp.float32
    bf = jnp.bfloat16
    fw = jnp.dot(rbf_ref[...].astype(bf), rbfw_ref[...].astype(bf),
                 preferred_element_type=f32)
    fo = gsc_ref[...][:, :HIDDEN] * fw
    msc_ref[...] = fo[:, NUM_IRREPS:]
    gt = fo[:, :NUM_IRREPS].T
    gsp = gsp_ref[...][:, :EDGE_DIM].T

    g0 = gt[:128, :]
    g1 = gt[128:192, :]
    g2 = gt[192:224, :]
    x0 = (gsp[:128, :] * g0).astype(bf)
    x1 = [(gsp[128 + 64 * i:128 + 64 * (i + 1), :] * g1).astype(bf)
          for i in range(3)]
    x2 = [(gsp[320 + 32 * i:320 + 32 * (i + 1), :] * g2).astype(bf)
          for i in range(5)]
    W0 = w0_ref[...].astype(bf)
    W1 = w1_ref[...].astype(bf)
    W2 = w2_ref[...].astype(bf)
    Y = {0: [jnp.dot(W0, x0, preferred_element_type=f32)],
         1: [jnp.dot(W1, x, preferred_element_type=f32) for x in x1],
         2: [jnp.dot(W2, x, preferred_element_type=f32) for x in x2]}

    rsh = rsht_ref[...]
    acc = {0: [None], 1: [None] * 3, 2: [None] * 5}
    for (l1, l2, io, yoff, mo, terms) in _COMBO:
        rbase = _RBASE[l2]
        for (i, k), jl in sorted(terms.items()):
            kv = None
            for (j, c) in jl:
                t = c * rsh[rbase + j:rbase + j + 1, :]
                kv = t if kv is None else kv + t
            contrib = kv * Y[l1][i][yoff:yoff + mo, :]
            acc[io][k] = contrib if acc[io][k] is None else acc[io][k] + contrib
    sph = jnp.concatenate(acc[0] + acc[1] + acc[2], axis=0)
    ms0_ref[...] = sph[0:128, :].T
    ms1_ref[...] = sph[128:256, :].T
    ms2_ref[...] = sph[256:384, :].T
    ms3_ref[...] = jnp.concatenate(
        [sph[384:480, :], jnp.zeros_like(sph[:32, :])], axis=0).T


def _edge_stage(gsc, gsp, rbf, rsh_t, rbf_w, W0t, W1t, W2t):
    n_edges = gsc.shape[0]
    nblk = n_edges // EDGE_BLK
    full = lambda shape: pl.BlockSpec(shape, lambda i: (0, 0))
    row = lambda n: pl.BlockSpec((EDGE_BLK, n), lambda i: (i, 0))
    col = lambda n: pl.BlockSpec((n, EDGE_BLK), lambda i: (0, i))
    return pl.pallas_call(
        _edge_body,
        grid=(nblk,),
        in_specs=[row(HID_P), row(SPH_P), row(NUM_BASIS), col(16),
                  full((NUM_BASIS, HIDDEN)), full((224, 128)),
                  full((384, 64)), full((352, 32))],
        out_specs=[row(128)] * 5,
        out_shape=[jax.ShapeDtypeStruct((n_edges, 128), jnp.float32)] * 5,
    )(gsc, gsp, rbf, rsh_t, rbf_w, W0t, W1t, W2t)


_SC_MESH = plsc.VectorSubcoreMesh(core_axis_name="c", subcore_axis_name="s",
                                  num_cores=2, num_subcores=16)
_NW = 32
_GC = 128
_NCHUNK = N_EDGES // _GC
HID_P = 384
SPH_P = 512


def _sc_gather(src_idx, scalar_tbl, sph_tbl):
    n_edges = src_idx.shape[0]
    nchunk = n_edges // _GC
    base_chunks = nchunk // _NW
    rem = nchunk % _NW

    @functools.partial(
        pl.kernel,
        out_type=[jax.ShapeDtypeStruct((n_edges, HID_P), jnp.float32),
                  jax.ShapeDtypeStruct((n_edges, SPH_P), jnp.float32)],
        mesh=_SC_MESH,
    )
    def gk(idx_hbm, t1_hbm, t2_hbm, o1_hbm, o2_hbm):
        wid = jax.lax.axis_index("s") * 2 + jax.lax.axis_index("c")
        nmine = base_chunks + jnp.where(wid < rem, 1, 0)

        def phase(tbl, out, width):
            def body(idx_v, rows_v, sem):
                def step(i, _):
                    e0 = (wid + i * _NW) * _GC
                    pltpu.sync_copy(idx_hbm.at[pl.ds(e0, _GC)], idx_v)
                    pltpu.async_copy(tbl.at[idx_v], rows_v, sem).wait()
                    pltpu.sync_copy(rows_v, out.at[pl.ds(e0, _GC)])
                    return 0
                jax.lax.fori_loop(0, nmine, step, 0)
            pl.run_scoped(body,
                          pltpu.VMEM((_GC,), jnp.int32),
                          pltpu.VMEM((_GC, width), jnp.float32),
                          pltpu.SemaphoreType.DMA)

        phase(t1_hbm, o1_hbm, HID_P)
        phase(t2_hbm, o2_hbm, SPH_P)

    return gk(src_idx, scalar_tbl, sph_tbl)


_SCHUNK = 128
_ROWS_PER_TILE = N_PAD // 16


def _sc_scatter(dst_idx, msgs, bases):
    nsuper = dst_idx.shape[0] // (2 * _SCHUNK)
    base_chunks = nsuper // 16
    rem = nsuper % 16
    dst2 = dst_idx.reshape(-1, _SCHUNK)

    @functools.partial(
        pl.kernel,
        out_type=[jax.ShapeDtypeStruct((N_PAD, 128), jnp.float32)] * 5,
        mesh=_SC_MESH,
        scratch_types=[pltpu.VMEM_SHARED((N_PAD, 128), jnp.float32)],
    )
    def sk(dst_hbm, m0, m1, m2, m3, m4, b0, b1, b2, b3, b4,
           o0, o1, o2, o3, o4, acc_s):
        cid = jax.lax.axis_index("c")
        sid = jax.lax.axis_index("s")
        r0 = sid * _ROWS_PER_TILE
        nmine = base_chunks + jnp.where(sid < rem, 1, 0)

        def chunk(m, b, o):
            pltpu.sync_copy(b.at[pl.ds(r0, _ROWS_PER_TILE)],
                            acc_s.at[pl.ds(r0, _ROWS_PER_TILE)])
            plsc.subcore_barrier()

            def body(idx_v, m_v, lsem, asem):
                def step(j, _):
                    sci = sid + j * 16
                    ca = pltpu.async_copy(dst_hbm.at[pl.ds(sci * 2, 2)],
                                          idx_v, lsem)
                    cb = pltpu.async_copy(m.at[pl.ds(sci * 2 * _SCHUNK,
                                                     2 * _SCHUNK)], m_v, lsem)
                    ca.wait()
                    cb.wait()
                    a0 = pltpu.async_copy(m_v.at[pl.ds(0, _SCHUNK)],
                                          acc_s.at[idx_v.at[0]], asem,
                                          add=True)
                    a1 = pltpu.async_copy(m_v.at[pl.ds(_SCHUNK, _SCHUNK)],
                                          acc_s.at[idx_v.at[1]], asem,
                                          add=True)
                    a0.wait()
                    a1.wait()
                    return 0
                jax.lax.fori_loop(0, nmine, step, 0)
            pl.run_scoped(body,
                          pltpu.VMEM((2, _SCHUNK), jnp.int32),
                          pltpu.VMEM((2 * _SCHUNK, 128), jnp.float32),
                          pltpu.SemaphoreType.DMA,
                          pltpu.SemaphoreType.DMA)
            plsc.subcore_barrier()
            pltpu.sync_copy(acc_s.at[pl.ds(r0, _ROWS_PER_TILE)],
                            o.at[pl.ds(r0, _ROWS_PER_TILE)])
            plsc.subcore_barrier()

        @pl.when(cid == 0)
        def _():
            chunk(m0, b0, o0)
            chunk(m1, b1, o1)
            chunk(m2, b2, o2)

        @pl.when(cid == 1)
        def _():
            chunk(m3, b3, o3)
            chunk(m4, b4, o4)

    return sk(dst2, *msgs, *bases)


def _to_imajor(sph):
    n = sph.shape[0]
    l1 = sph[:, 128:320].reshape(n, 64, 3).transpose(0, 2, 1).reshape(n, 192)
    l2 = sph[:, 320:480].reshape(n, 32, 5).transpose(0, 2, 1).reshape(n, 160)
    return jnp.concatenate([sph[:, :128], l1, l2], axis=1)


def _from_imajor(sph):
    n = sph.shape[0]
    l1 = sph[:, 128:320].reshape(n, 3, 64).transpose(0, 2, 1).reshape(n, 192)
    l2 = sph[:, 320:480].reshape(n, 5, 32).transpose(0, 2, 1).reshape(n, 160)
    return jnp.concatenate([sph[:, :128], l1, l2], axis=1)


def kernel(x_scalar, x_spherical, rbf, rsh, W1, b1, W2, b2, rbf_w, ln_g, ln_b,
           o3_w, o3_b, tp_w, edge_index):
    col_w = jnp.concatenate([
        o3_w[:128],
        jnp.repeat(o3_w[128:192], 3),
        jnp.repeat(o3_w[192:224], 5)]).reshape(1, EDGE_DIM)
    col_b = jnp.concatenate(
        [o3_b, jnp.zeros((EDGE_DIM - 128,), jnp.float32)]).reshape(1, EDGE_DIM)

    xs = jnp.pad(x_scalar, ((0, N_PAD - N_NODES), (0, 0)))
    xsp = jnp.pad(x_spherical, ((0, N_PAD - N_NODES), (0, 0)))
    scalar_in, scalar_out, sph_in = _node_stage(
        xs, xsp, W1, b1, W2, b2, ln_g, ln_b, col_w, col_b)
    sph_in_im = _to_imajor(sph_in)

    W0c, W1c, W2c = _prep_tp_weights(tp_w)

    src = edge_index[1]
    dst = edge_index[0]
    sc_tbl = jnp.pad(scalar_out, ((0, 0), (0, HID_P - HIDDEN)))
    sp_tbl = jnp.pad(sph_in_im, ((0, 0), (0, SPH_P - EDGE_DIM)))
    rsh_t = jnp.pad(rsh.T, ((0, 16 - SPH_DIM), (0, 0)))

    cuts = [0, 40192, 80128, 120064, N_EDGES]
    parts = list(zip(cuts[:-1], cuts[1:]))
    gathered = [_sc_gather(src[lo:hi], sc_tbl, sp_tbl) for (lo, hi) in parts]
    tables = (scalar_in, sp_tbl[:, 0:128], sp_tbl[:, 128:256],
              sp_tbl[:, 256:384], sp_tbl[:, 384:512])
    for (lo, hi), (gsc, gsp) in zip(parts, gathered):
        msgs = _edge_stage(gsc, gsp, rbf[lo:hi], rsh_t[:, lo:hi], rbf_w,
                           W0c.T, W1c.T, W2c.T)
        tables = _sc_scatter(dst[lo:hi], msgs, tables)
    new_scalar = tables[0][:N_NODES]
    new_sph_im = jnp.concatenate(tables[1:], axis=1)[:N_NODES, :EDGE_DIM]
    return new_scalar, _from_imajor(new_sph_im)
